# Initial kernel scaffold; baseline (speedup 1.0000x reference)
import functools

import jax
import jax.numpy as jnp
from jax import lax
from jax.experimental import pallas as pl
from jax.experimental.pallas import tpu as pltpu
from jax.experimental.pallas import tpu_sc as plsc

N = 10000
B = 64
E_TOT = 330000
NC, NS, LANES = 2, 16, 16
NW = NC * NS
EP = 330240
PER_W = EP // NW
C = 240
NCH = PER_W // C
NG = C // LANES
ROWS_T = N // NS

F32 = jnp.float32
I32 = jnp.int32


def _sc_edge_pass(xl, xr, src, dst, att, D):
    mesh = plsc.VectorSubcoreMesh(
        core_axis_name="c", subcore_axis_name="s",
        num_cores=NC, num_subcores=NS)

    @functools.partial(
        pl.kernel,
        out_type=[jax.ShapeDtypeStruct((NC, N, D), F32),
                  jax.ShapeDtypeStruct((NC, N), F32)],
        mesh=mesh,
        scratch_types=[
            pltpu.VMEM_SHARED((N, D), F32),
            pltpu.VMEM_SHARED((N,), F32),
            pltpu.VMEM((C,), I32),
            pltpu.VMEM((C,), I32),
            pltpu.VMEM((C, D), F32),
            pltpu.VMEM((C, D), F32),
            pltpu.VMEM((C,), F32),
            pltpu.VMEM((D,), F32),
            pltpu.VMEM((1024,), F32),
            pltpu.SemaphoreType.DMA,
            pltpu.SemaphoreType.DMA,
        ],
    )
    def k(xl_h, xr_h, src_h, dst_h, att_h, out_h, den_h,
          acc_sh, den_sh, src_v, dst_v, bufL, bufR, exb, att_v, zrow,
          seml, semr):
        cid = lax.axis_index("c")
        sid = lax.axis_index("s")
        wid = sid * NC + cid
        zero16 = jnp.zeros((LANES,), F32)
        iota = lax.iota(I32, LANES)
        rows = [g * LANES + iota for g in range(NG)]

        def zz(i, c):
            zrow[pl.ds(i * LANES, LANES)] = zero16
            return c
        lax.fori_loop(0, 1024 // LANES, zz, 0)

        def zb(r, c):
            for ko in range(D // LANES):
                bufL[r, pl.ds(ko * LANES, LANES)] = zero16
            return c
        lax.fori_loop(0, C, zb, 0)

        r0 = sid * ROWS_T
        for off, r in ((0, 240), (240, 240), (480, 145)):
            pltpu.sync_copy(bufL.at[pl.ds(0, r)],
                            acc_sh.at[pl.ds(r0 + off, r)])

        @pl.when(sid == 0)
        def _zd():
            for kk in range(N // 1000):
                pltpu.sync_copy(zrow.at[pl.ds(0, 1000)],
                                den_sh.at[pl.ds(kk * 1000, 1000)])

        pltpu.sync_copy(att_h, att_v)
        plsc.subcore_barrier()

        def chunk(ci, carry):
            base = pl.multiple_of(wid * PER_W + ci * C, 8)
            pltpu.sync_copy(src_h.at[pl.ds(base, C)], src_v)
            pltpu.sync_copy(dst_h.at[pl.ds(base, C)], dst_v)
            cl = pltpu.async_copy(xl_h.at[src_v], bufL, seml)
            cr = pltpu.async_copy(xr_h.at[dst_v], bufR, semr)
            cl.wait()
            cr.wait()

            def fA(f, accs):
                fs = jnp.full((LANES,), f, I32)
                av = plsc.load_gather(att_v, [fs])
                out = []
                for g in range(NG):
                    a = plsc.load_gather(bufL, [rows[g], fs])
                    b = plsc.load_gather(bufR, [rows[g], fs])
                    h = a + b
                    h = jnp.where(h > 0, h, 0.2 * h)
                    out.append(accs[g] + h * av)
                return tuple(out)
            accs = lax.fori_loop(0, D, fA,
                                 tuple(zero16 for _ in range(NG)))

            exs = []
            for g in range(NG):
                gid = base + rows[g]
                ex = jnp.where(gid < E_TOT, jnp.exp(accs[g]), 0.0)
                exb[pl.ds(g * LANES, LANES)] = ex
                exs.append(ex)

            def fB(f, c):
                fs = jnp.full((LANES,), f, I32)
                for g in range(NG):
                    v = plsc.load_gather(bufL, [rows[g], fs])
                    plsc.store_scatter(bufL, [rows[g], fs], v * exs[g])
                return c
            lax.fori_loop(0, D, fB, 0)

            pltpu.sync_copy(bufL, acc_sh.at[dst_v], add=True)
            pltpu.sync_copy(exb, den_sh.at[dst_v], add=True)
            return carry
        lax.fori_loop(0, NCH, chunk, 0)

        plsc.subcore_barrier()

        for off, r in ((0, 240), (240, 240), (480, 145)):
            pltpu.sync_copy(acc_sh.at[pl.ds(r0 + off, r)],
                            out_h.at[cid, pl.ds(r0 + off, r)])

        @pl.when(sid == 0)
        def _wd():
            pltpu.sync_copy(den_sh, den_h.at[cid])

    return k(xl, xr, src, dst, att)


def _tc_dual_matmul(x, Wl, Wr, K, M):
    R = 400

    def body(x_ref, wl_ref, wr_ref, ol_ref, or_ref):
        xb = x_ref[...]
        ol_ref[...] = jnp.dot(xb, wl_ref[...], preferred_element_type=F32)
        or_ref[...] = jnp.dot(xb, wr_ref[...], preferred_element_type=F32)

    return pl.pallas_call(
        body,
        grid=(N // R,),
        in_specs=[pl.BlockSpec((R, K), lambda i: (i, 0)),
                  pl.BlockSpec((K, M), lambda i: (0, 0)),
                  pl.BlockSpec((K, M), lambda i: (0, 0))],
        out_specs=[pl.BlockSpec((R, M), lambda i: (i, 0)),
                   pl.BlockSpec((R, M), lambda i: (i, 0))],
        out_shape=[jax.ShapeDtypeStruct((N, M), F32),
                   jax.ShapeDtypeStruct((N, M), F32)],
    )(x, Wl, Wr)


def _tc_norm_elu_matmul(p, den, bias, Wl, Wr, K, M):
    R = 400
    den3 = den.reshape(NC, N, 1)
    b2 = bias.reshape(1, K)

    def body(p_ref, d_ref, b_ref, wl_ref, wr_ref, ol_ref, or_ref):
        h = p_ref[0] + p_ref[1]
        dd = d_ref[0] + d_ref[1]
        h = h / dd + b_ref[...]
        h = jnp.where(h > 0, h, jnp.exp(jnp.minimum(h, 0.0)) - 1.0)
        ol_ref[...] = jnp.dot(h, wl_ref[...], preferred_element_type=F32)
        or_ref[...] = jnp.dot(h, wr_ref[...], preferred_element_type=F32)

    return pl.pallas_call(
        body,
        grid=(N // R,),
        in_specs=[pl.BlockSpec((NC, R, K), lambda i: (0, i, 0)),
                  pl.BlockSpec((NC, R, 1), lambda i: (0, i, 0)),
                  pl.BlockSpec((1, K), lambda i: (0, 0)),
                  pl.BlockSpec((K, M), lambda i: (0, 0)),
                  pl.BlockSpec((K, M), lambda i: (0, 0))],
        out_specs=[pl.BlockSpec((R, M), lambda i: (i, 0)),
                   pl.BlockSpec((R, M), lambda i: (i, 0))],
        out_shape=[jax.ShapeDtypeStruct((N, M), F32),
                   jax.ShapeDtypeStruct((N, M), F32)],
    )(p, den3, b2, Wl, Wr)


def _tc_final(p2, den2, b2, batch, obs, nf, ne,
              oW1, ob1, oW2, ob2, nW, nb, fW1, fb1, fW2, fb2):
    den3 = den2.reshape(NC, N, 1)
    bt2 = batch.reshape(N, 1)
    fa, fb, fc, fd = fW1[:32], fW1[32:40], fW1[40:44], fW1[44:45]

    def body(p_ref, d_ref, b_ref, bt_ref, obs_ref, nf_ref, ne_ref,
             ow1, ob1r, ow2, ob2r, nwr, nbr,
             far, fbr, fcr, fdr, fb1r, fw2r, fb2r, o_ref):
        h = p_ref[0] + p_ref[1]
        dd = d_ref[0] + d_ref[1]
        h = h / dd + b_ref[...]
        h = jnp.where(h > 0, h, jnp.exp(jnp.minimum(h, 0.0)) - 1.0)
        bt = bt_ref[...]
        seg = lax.broadcasted_iota(I32, (B, N), 0)
        oh = (seg == bt.reshape(1, N)).astype(F32)
        sums = jnp.dot(oh, h, preferred_element_type=F32)
        cnt = jnp.sum(oh, axis=1, keepdims=True)
        ce = sums / jnp.maximum(cnt, 1.0)
        obs_e = jnp.dot(
            jnp.maximum(jnp.dot(obs_ref[...], ow1[...],
                                preferred_element_type=F32) + ob1r[...], 0.0),
            ow2[...], preferred_element_type=F32) + ob2r[...]
        noi = jnp.dot(nf_ref[...], nwr[...],
                      preferred_element_type=F32) + nbr[...]
        z = (jnp.dot(ce, far[...], preferred_element_type=F32)
             + jnp.dot(obs_e, fbr[...], preferred_element_type=F32)
             + jnp.dot(noi, fcr[...], preferred_element_type=F32)
             + jnp.dot(ne_ref[...], fdr[...], preferred_element_type=F32)
             + fb1r[...])
        corr = jnp.dot(jnp.maximum(z, 0.0), fw2r[...],
                       preferred_element_type=F32) + fb2r[...]
        o_ref[...] = ne_ref[...] + corr

    args = (p2, den3, b2.reshape(1, 32), bt2, obs, nf, ne,
            oW1, ob1.reshape(1, 32), oW2, ob2.reshape(1, 8),
            nW, nb.reshape(1, 4),
            fa, fb, fc, fd, fb1.reshape(1, 256), fW2, fb2.reshape(1, 1))
    return pl.pallas_call(
        body,
        out_shape=jax.ShapeDtypeStruct((B, 1), F32),
    )(*args)


def kernel(x, edge_index, batch, observable_features, noise_factor,
           noisy_exp, Wl1, Wr1, att1, b1, Wl2, Wr2, att2, b2,
           obs_W1, obs_b1, obs_W2, obs_b2, noise_W, noise_b,
           fus_W1, fus_b1, fus_W2, fus_b2):
    loop = jnp.arange(N, dtype=edge_index.dtype)
    pad = jnp.zeros((EP - E_TOT,), edge_index.dtype)
    src = jnp.concatenate([edge_index[0], loop, pad])
    dst = jnp.concatenate([edge_index[1], loop, pad])

    xl1, xr1 = _tc_dual_matmul(x, Wl1, Wr1, 128, 128)
    p1, d1 = _sc_edge_pass(xl1, xr1, src, dst, att1, 128)
    xl2, xr2 = _tc_norm_elu_matmul(p1, d1, b1, Wl2, Wr2, 128, 32)
    p2, d2 = _sc_edge_pass(xl2, xr2, src, dst, att2, 32)
    obs = observable_features.reshape(B, 5)
    return _tc_final(p2, d2, b2, batch, obs, noise_factor, noisy_exp,
                     obs_W1, obs_b1, obs_W2, obs_b2, noise_W, noise_b,
                     fus_W1, fus_b1, fus_W2, fus_b2)


# speedup vs baseline: 3.2593x; 3.2593x over previous
import functools

import jax
import jax.numpy as jnp
from jax import lax
from jax.experimental import pallas as pl
from jax.experimental.pallas import tpu as pltpu
from jax.experimental.pallas import tpu_sc as plsc

N = 10000
B = 64
E_TOT = 330000
NC, NS, LANES = 2, 16, 16
NW = NC * NS
EP = 331776
PER_W = EP // NW
C = 128
NCH = PER_W // C
NG = C // LANES
ROWS_T = 624

F32 = jnp.float32
I32 = jnp.int32


def _sc_edge_pass(xl, xr, src, dst, att, D):
    mesh = plsc.VectorSubcoreMesh(
        core_axis_name="c", subcore_axis_name="s",
        num_cores=NC, num_subcores=NS)

    @functools.partial(
        pl.kernel,
        out_type=[jax.ShapeDtypeStruct((NC, N, D), F32),
                  jax.ShapeDtypeStruct((NC, N), F32)],
        mesh=mesh,
        compiler_params=pltpu.CompilerParams(
            needs_layout_passes=False,
            use_tc_tiling_on_sc=(D % 128 == 0)),
        scratch_types=[
            pltpu.VMEM_SHARED((N, D), F32),
            pltpu.VMEM_SHARED((N,), F32),
            pltpu.VMEM((C,), I32),
            pltpu.VMEM((C,), I32),
            pltpu.VMEM((C, D), F32),
            pltpu.VMEM((C, D), F32),
            pltpu.VMEM((C,), F32),
            pltpu.VMEM((D,), F32),
            pltpu.VMEM((1024,), F32),
            pltpu.SemaphoreType.DMA,
            pltpu.SemaphoreType.DMA,
        ],
    )
    def k(xl_h, xr_h, src_h, dst_h, att_h, out_h, den_h,
          acc_sh, den_sh, src_v, dst_v, bufL, bufR, exb, att_v, zrow,
          seml, semr):
        cid = lax.axis_index("c")
        sid = lax.axis_index("s")
        wid = sid * NC + cid
        zero16 = jnp.zeros((LANES,), F32)
        iota = lax.iota(I32, LANES)
        rows = [g * LANES + iota for g in range(NG)]

        def zz(i, c):
            zrow[pl.ds(i * LANES, LANES)] = zero16
            return c
        lax.fori_loop(0, 1024 // LANES, zz, 0)

        def zb(r, c):
            for ko in range(D // LANES):
                bufL[r, pl.ds(ko * LANES, LANES)] = zero16
            return c
        lax.fori_loop(0, C, zb, 0)

        r0 = sid * ROWS_T
        for off in range(0, ROWS_T, C):
            r = min(C, ROWS_T - off)
            pltpu.sync_copy(bufL.at[pl.ds(0, r)],
                            acc_sh.at[pl.ds(r0 + off, r)])

        @pl.when(sid == NS - 1)
        def _ztail():
            pltpu.sync_copy(bufL.at[pl.ds(0, 16)],
                            acc_sh.at[pl.ds(N - 16, 16)])

        @pl.when(sid == 0)
        def _zd():
            for kk in range(N // 1000):
                pltpu.sync_copy(zrow.at[pl.ds(0, 1000)],
                                den_sh.at[pl.ds(kk * 1000, 1000)])

        pltpu.sync_copy(att_h, att_v)
        plsc.subcore_barrier()

        def chunk(ci, carry):
            base = pl.multiple_of(wid * PER_W + ci * C, 8)
            pltpu.sync_copy(src_h.at[pl.ds(base, C)], src_v)
            pltpu.sync_copy(dst_h.at[pl.ds(base, C)], dst_v)
            cl = pltpu.async_copy(xl_h.at[src_v], bufL, seml)
            cr = pltpu.async_copy(xr_h.at[dst_v], bufR, semr)
            cl.wait()
            cr.wait()

            def fA(f, accs):
                fs = jnp.full((LANES,), f, I32)
                av = plsc.load_gather(att_v, [fs])
                out = []
                for g in range(NG):
                    a = plsc.load_gather(bufL, [rows[g], fs])
                    b = plsc.load_gather(bufR, [rows[g], fs])
                    h = a + b
                    h = jnp.where(h > 0, h, 0.2 * h)
                    out.append(accs[g] + h * av)
                return tuple(out)
            accs = lax.fori_loop(0, D, fA,
                                 tuple(zero16 for _ in range(NG)))

            exs = []
            for g in range(NG):
                gid = base + rows[g]
                ex = jnp.where(gid < E_TOT, jnp.exp(accs[g]), 0.0)
                exb[pl.ds(g * LANES, LANES)] = ex
                exs.append(ex)

            def fB(f, c):
                fs = jnp.full((LANES,), f, I32)
                for g in range(NG):
                    v = plsc.load_gather(bufL, [rows[g], fs])
                    plsc.store_scatter(bufL, [rows[g], fs], v * exs[g])
                return c
            lax.fori_loop(0, D, fB, 0)

            pltpu.sync_copy(bufL, acc_sh.at[dst_v], add=True)
            pltpu.sync_copy(exb, den_sh.at[dst_v], add=True)
            return carry
        lax.fori_loop(0, NCH, chunk, 0)

        plsc.subcore_barrier()

        pltpu.sync_copy(acc_sh.at[pl.ds(r0, ROWS_T)],
                        out_h.at[cid, pl.ds(r0, ROWS_T)])

        @pl.when(sid == NS - 1)
        def _wtail():
            pltpu.sync_copy(acc_sh.at[pl.ds(N - 16, 16)],
                            out_h.at[cid, pl.ds(N - 16, 16)])

        @pl.when(sid == 0)
        def _wd():
            pltpu.sync_copy(den_sh, den_h.at[cid])

    return k(xl, xr, src, dst, att)


def _tc_dual_matmul(x, Wl, Wr, K, M):
    R = 400

    def body(x_ref, wl_ref, wr_ref, ol_ref, or_ref):
        xb = x_ref[...]
        ol_ref[...] = jnp.dot(xb, wl_ref[...], preferred_element_type=F32)
        or_ref[...] = jnp.dot(xb, wr_ref[...], preferred_element_type=F32)

    return pl.pallas_call(
        body,
        grid=(N // R,),
        in_specs=[pl.BlockSpec((R, K), lambda i: (i, 0)),
                  pl.BlockSpec((K, M), lambda i: (0, 0)),
                  pl.BlockSpec((K, M), lambda i: (0, 0))],
        out_specs=[pl.BlockSpec((R, M), lambda i: (i, 0)),
                   pl.BlockSpec((R, M), lambda i: (i, 0))],
        out_shape=[jax.ShapeDtypeStruct((N, M), F32),
                   jax.ShapeDtypeStruct((N, M), F32)],
    )(x, Wl, Wr)


def _tc_norm_elu_matmul(p, den, bias, Wl, Wr, K, M):
    R = 400
    den3 = den.reshape(NC, N, 1)
    b2 = bias.reshape(1, K)

    def body(p_ref, d_ref, b_ref, wl_ref, wr_ref, ol_ref, or_ref):
        h = p_ref[0] + p_ref[1]
        dd = d_ref[0] + d_ref[1]
        h = h / dd + b_ref[...]
        h = jnp.where(h > 0, h, jnp.exp(jnp.minimum(h, 0.0)) - 1.0)
        ol_ref[...] = jnp.dot(h, wl_ref[...], preferred_element_type=F32)
        or_ref[...] = jnp.dot(h, wr_ref[...], preferred_element_type=F32)

    return pl.pallas_call(
        body,
        grid=(N // R,),
        in_specs=[pl.BlockSpec((NC, R, K), lambda i: (0, i, 0)),
                  pl.BlockSpec((NC, R, 1), lambda i: (0, i, 0)),
                  pl.BlockSpec((1, K), lambda i: (0, 0)),
                  pl.BlockSpec((K, M), lambda i: (0, 0)),
                  pl.BlockSpec((K, M), lambda i: (0, 0))],
        out_specs=[pl.BlockSpec((R, M), lambda i: (i, 0)),
                   pl.BlockSpec((R, M), lambda i: (i, 0))],
        out_shape=[jax.ShapeDtypeStruct((N, M), F32),
                   jax.ShapeDtypeStruct((N, M), F32)],
    )(p, den3, b2, Wl, Wr)


def _tc_final(p2, den2, b2, batch, obs, nf, ne,
              oW1, ob1, oW2, ob2, nW, nb, fW1, fb1, fW2, fb2):
    den3 = den2.reshape(NC, N, 1)
    bt2 = batch.reshape(N, 1)
    fa, fb, fc, fd = fW1[:32], fW1[32:40], fW1[40:44], fW1[44:45]

    def body(p_ref, d_ref, b_ref, bt_ref, obs_ref, nf_ref, ne_ref,
             ow1, ob1r, ow2, ob2r, nwr, nbr,
             far, fbr, fcr, fdr, fb1r, fw2r, fb2r, o_ref):
        h = p_ref[0] + p_ref[1]
        dd = d_ref[0] + d_ref[1]
        h = h / dd + b_ref[...]
        h = jnp.where(h > 0, h, jnp.exp(jnp.minimum(h, 0.0)) - 1.0)
        bt = bt_ref[...]
        seg = lax.broadcasted_iota(I32, (B, N), 0)
        oh = (seg == bt.reshape(1, N)).astype(F32)
        sums = jnp.dot(oh, h, preferred_element_type=F32)
        cnt = jnp.sum(oh, axis=1, keepdims=True)
        ce = sums / jnp.maximum(cnt, 1.0)
        obs_e = jnp.dot(
            jnp.maximum(jnp.dot(obs_ref[...], ow1[...],
                                preferred_element_type=F32) + ob1r[...], 0.0),
            ow2[...], preferred_element_type=F32) + ob2r[...]
        noi = jnp.dot(nf_ref[...], nwr[...],
                      preferred_element_type=F32) + nbr[...]
        z = (jnp.dot(ce, far[...], preferred_element_type=F32)
             + jnp.dot(obs_e, fbr[...], preferred_element_type=F32)
             + jnp.dot(noi, fcr[...], preferred_element_type=F32)
             + jnp.dot(ne_ref[...], fdr[...], preferred_element_type=F32)
             + fb1r[...])
        corr = jnp.dot(jnp.maximum(z, 0.0), fw2r[...],
                       preferred_element_type=F32) + fb2r[...]
        o_ref[...] = ne_ref[...] + corr

    args = (p2, den3, b2.reshape(1, 32), bt2, obs, nf, ne,
            oW1, ob1.reshape(1, 32), oW2, ob2.reshape(1, 8),
            nW, nb.reshape(1, 4),
            fa, fb, fc, fd, fb1.reshape(1, 256), fW2, fb2.reshape(1, 1))
    return pl.pallas_call(
        body,
        out_shape=jax.ShapeDtypeStruct((B, 1), F32),
    )(*args)


def kernel(x, edge_index, batch, observable_features, noise_factor,
           noisy_exp, Wl1, Wr1, att1, b1, Wl2, Wr2, att2, b2,
           obs_W1, obs_b1, obs_W2, obs_b2, noise_W, noise_b,
           fus_W1, fus_b1, fus_W2, fus_b2):
    loop = jnp.arange(N, dtype=edge_index.dtype)
    pad = jnp.zeros((EP - E_TOT,), edge_index.dtype)
    src = jnp.concatenate([edge_index[0], loop, pad])
    dst = jnp.concatenate([edge_index[1], loop, pad])

    xl1, xr1 = _tc_dual_matmul(x, Wl1, Wr1, 128, 128)
    p1, d1 = _sc_edge_pass(xl1, xr1, src, dst, att1, 128)
    xl2, xr2 = _tc_norm_elu_matmul(p1, d1, b1, Wl2, Wr2, 128, 32)
    p2, d2 = _sc_edge_pass(xl2, xr2, src, dst, att2, 32)
    obs = observable_features.reshape(B, 5)
    return _tc_final(p2, d2, b2, batch, obs, noise_factor, noisy_exp,
                     obs_W1, obs_b1, obs_W2, obs_b2, noise_W, noise_b,
                     fus_W1, fus_b1, fus_W2, fus_b2)


# speedup vs baseline: 3.8643x; 1.1856x over previous
import functools

import jax
import jax.numpy as jnp
from jax import lax
from jax.experimental import pallas as pl
from jax.experimental.pallas import tpu as pltpu
from jax.experimental.pallas import tpu_sc as plsc

N = 10000
B = 64
E_TOT = 330000
NC, NS, LANES = 2, 16, 16
NW = NC * NS
EP = 331776
PER_W = EP // NW
C = 48
NCH = PER_W // C
NG = C // LANES
ROWS_T = 624

F32 = jnp.float32
I32 = jnp.int32


def _sc_edge_pass(xl, xr, src, dst, att, D):
    mesh = plsc.VectorSubcoreMesh(
        core_axis_name="c", subcore_axis_name="s",
        num_cores=NC, num_subcores=NS)

    @functools.partial(
        pl.kernel,
        out_type=[jax.ShapeDtypeStruct((NC, N, D), F32),
                  jax.ShapeDtypeStruct((NC, N), F32)],
        mesh=mesh,
        compiler_params=pltpu.CompilerParams(
            needs_layout_passes=False,
            use_tc_tiling_on_sc=(D % 128 == 0)),
        scratch_types=[
            pltpu.VMEM_SHARED((N, D), F32),
            pltpu.VMEM_SHARED((N,), F32),
            pltpu.VMEM((2, C), I32),
            pltpu.VMEM((2, C), I32),
            pltpu.VMEM((2, C, D), F32),
            pltpu.VMEM((2, C, D), F32),
            pltpu.VMEM((2, C, D), F32),
            pltpu.VMEM((2, C), F32),
            pltpu.VMEM((2, C), I32),
            pltpu.VMEM((D,), F32),
            pltpu.VMEM((1024,), F32),
            [pltpu.SemaphoreType.DMA] * 2,
            [pltpu.SemaphoreType.DMA] * 2,
            [pltpu.SemaphoreType.DMA] * 2,
        ],
    )
    def k(xl_h, xr_h, src_h, dst_h, att_h, out_h, den_h,
          acc_sh, den_sh, ibs, ibd, bufL, bufR, bufS, exb, sidx, att_v,
          zrow, semG, semS, semI):
        cid = lax.axis_index("c")
        sid = lax.axis_index("s")
        wid = sid * NC + cid
        zero16 = jnp.zeros((LANES,), F32)
        iota = lax.iota(I32, LANES)
        rows = [g * LANES + iota for g in range(NG)]
        cbase = wid * NCH

        def zz(i, c):
            zrow[pl.ds(i * LANES, LANES)] = zero16
            return c
        lax.fori_loop(0, 1024 // LANES, zz, 0)

        def zb(r, c):
            for ko in range(D // LANES):
                bufS[0, r, pl.ds(ko * LANES, LANES)] = zero16
            return c
        lax.fori_loop(0, C, zb, 0)

        r0 = sid * ROWS_T
        for off in range(0, ROWS_T, C):
            r = min(C, ROWS_T - off)
            pltpu.sync_copy(bufS.at[0, pl.ds(0, r)],
                            acc_sh.at[pl.ds(r0 + off, r)])

        @pl.when(sid == NS - 1)
        def _ztail():
            pltpu.sync_copy(bufS.at[0, pl.ds(0, 16)],
                            acc_sh.at[pl.ds(N - 16, 16)])

        @pl.when(sid == 0)
        def _zd():
            for kk in range(N // 1000):
                pltpu.sync_copy(zrow.at[pl.ds(0, 1000)],
                                den_sh.at[pl.ds(kk * 1000, 1000)])

        pltpu.sync_copy(att_h, att_v)
        plsc.subcore_barrier()

        def issue_idx(i, p):
            pltpu.async_copy(src_h.at[cbase + i], ibs.at[p], semI[p])
            pltpu.async_copy(dst_h.at[cbase + i], ibd.at[p], semI[p])

        def wait_idx(p):
            pltpu.make_async_copy(src_h.at[cbase], ibs.at[p],
                                  semI[p]).wait()
            pltpu.make_async_copy(dst_h.at[cbase], ibd.at[p],
                                  semI[p]).wait()

        def issue_rows(p):
            pltpu.async_copy(xl_h.at[ibs.at[p]], bufL.at[p], semG[p])
            pltpu.async_copy(xr_h.at[ibd.at[p]], bufR.at[p], semG[p])

        def wait_rows(p):
            pltpu.make_async_copy(xl_h.at[ibs.at[p]], bufL.at[p],
                                  semG[p]).wait()
            pltpu.make_async_copy(xr_h.at[ibd.at[p]], bufR.at[p],
                                  semG[p]).wait()

        def issue_scat(p):
            pltpu.async_copy(bufS.at[p], acc_sh.at[sidx.at[p]],
                             semS[p], add=True)
            pltpu.async_copy(exb.at[p], den_sh.at[sidx.at[p]],
                             semS[p], add=True)

        def wait_scat(p):
            pltpu.make_async_copy(bufS.at[p], acc_sh.at[sidx.at[p]],
                                  semS[p]).wait()
            pltpu.make_async_copy(exb.at[p], den_sh.at[sidx.at[p]],
                                  semS[p]).wait()

        def do_chunk(i, p, wait_i, wait_s):
            wait_rows(p)
            if wait_s:
                wait_scat(p)
            for g in range(NG):
                sidx[p, pl.ds(g * LANES, LANES)] = (
                    ibd[p, pl.ds(g * LANES, LANES)])

            @pl.when(i + 2 < NCH)
            def _():
                issue_idx(i + 2, p)
            if wait_i:
                @pl.when(i + 1 < NCH)
                def _():
                    wait_idx(1 - p)
                    issue_rows(1 - p)
            else:
                issue_rows(1 - p)

            def fA(f, accs):
                fs = jnp.full((LANES,), f, I32)
                av = plsc.load_gather(att_v, [fs])
                out = []
                for g in range(NG):
                    a = plsc.load_gather(bufL.at[p], [rows[g], fs])
                    b = plsc.load_gather(bufR.at[p], [rows[g], fs])
                    h = a + b
                    h = jnp.where(h > 0, h, 0.2 * h)
                    out.append(accs[g] + h * av)
                return tuple(out)
            accs = lax.fori_loop(0, D, fA,
                                 tuple(zero16 for _ in range(NG)),
                                 unroll=4)

            base_e = (cbase + i) * C
            for g in range(NG):
                gid = base_e + rows[g]
                ex = jnp.where(gid < E_TOT, jnp.exp(accs[g]), 0.0)
                exb[p, pl.ds(g * LANES, LANES)] = ex

            exs = [exb[p, pl.ds(g * LANES, LANES)] for g in range(NG)]

            def fB(f, c):
                fs = jnp.full((LANES,), f, I32)
                for g in range(NG):
                    v = plsc.load_gather(bufL.at[p], [rows[g], fs])
                    plsc.store_scatter(bufS.at[p], [rows[g], fs],
                                       v * exs[g])
                return c
            lax.fori_loop(0, D, fB, 0, unroll=4)
            issue_scat(p)

        for p in (0, 1):
            issue_idx(p, p)
            wait_idx(p)
        issue_rows(0)
        do_chunk(0, 0, wait_i=False, wait_s=False)
        do_chunk(1, 1, wait_i=True, wait_s=False)

        def pair(j, c):
            do_chunk(2 * j, 0, wait_i=True, wait_s=True)
            do_chunk(2 * j + 1, 1, wait_i=True, wait_s=True)
            return c
        lax.fori_loop(1, NCH // 2, pair, 0)

        wait_scat(0)
        wait_scat(1)

        plsc.subcore_barrier()

        pltpu.sync_copy(acc_sh.at[pl.ds(r0, ROWS_T)],
                        out_h.at[cid, pl.ds(r0, ROWS_T)])

        @pl.when(sid == NS - 1)
        def _wtail():
            pltpu.sync_copy(acc_sh.at[pl.ds(N - 16, 16)],
                            out_h.at[cid, pl.ds(N - 16, 16)])

        @pl.when(sid == 0)
        def _wd():
            pltpu.sync_copy(den_sh, den_h.at[cid])

    return k(xl, xr, src.reshape(EP // C, C), dst.reshape(EP // C, C),
             att)


def _tc_dual_matmul(x, Wl, Wr, K, M):
    R = 400

    def body(x_ref, wl_ref, wr_ref, ol_ref, or_ref):
        xb = x_ref[...]
        ol_ref[...] = jnp.dot(xb, wl_ref[...], preferred_element_type=F32)
        or_ref[...] = jnp.dot(xb, wr_ref[...], preferred_element_type=F32)

    return pl.pallas_call(
        body,
        grid=(N // R,),
        in_specs=[pl.BlockSpec((R, K), lambda i: (i, 0)),
                  pl.BlockSpec((K, M), lambda i: (0, 0)),
                  pl.BlockSpec((K, M), lambda i: (0, 0))],
        out_specs=[pl.BlockSpec((R, M), lambda i: (i, 0)),
                   pl.BlockSpec((R, M), lambda i: (i, 0))],
        out_shape=[jax.ShapeDtypeStruct((N, M), F32),
                   jax.ShapeDtypeStruct((N, M), F32)],
    )(x, Wl, Wr)


def _tc_norm_elu_matmul(p, den, bias, Wl, Wr, K, M):
    R = 400
    den3 = den.reshape(NC, N, 1)
    b2 = bias.reshape(1, K)

    def body(p_ref, d_ref, b_ref, wl_ref, wr_ref, ol_ref, or_ref):
        h = p_ref[0] + p_ref[1]
        dd = d_ref[0] + d_ref[1]
        h = h / dd + b_ref[...]
        h = jnp.where(h > 0, h, jnp.exp(jnp.minimum(h, 0.0)) - 1.0)
        ol_ref[...] = jnp.dot(h, wl_ref[...], preferred_element_type=F32)
        or_ref[...] = jnp.dot(h, wr_ref[...], preferred_element_type=F32)

    return pl.pallas_call(
        body,
        grid=(N // R,),
        in_specs=[pl.BlockSpec((NC, R, K), lambda i: (0, i, 0)),
                  pl.BlockSpec((NC, R, 1), lambda i: (0, i, 0)),
                  pl.BlockSpec((1, K), lambda i: (0, 0)),
                  pl.BlockSpec((K, M), lambda i: (0, 0)),
                  pl.BlockSpec((K, M), lambda i: (0, 0))],
        out_specs=[pl.BlockSpec((R, M), lambda i: (i, 0)),
                   pl.BlockSpec((R, M), lambda i: (i, 0))],
        out_shape=[jax.ShapeDtypeStruct((N, M), F32),
                   jax.ShapeDtypeStruct((N, M), F32)],
    )(p, den3, b2, Wl, Wr)


def _tc_final(p2, den2, b2, batch, obs, nf, ne,
              oW1, ob1, oW2, ob2, nW, nb, fW1, fb1, fW2, fb2):
    den3 = den2.reshape(NC, N, 1)
    bt2 = batch.reshape(N, 1)
    fa, fb, fc, fd = fW1[:32], fW1[32:40], fW1[40:44], fW1[44:45]

    def body(p_ref, d_ref, b_ref, bt_ref, obs_ref, nf_ref, ne_ref,
             ow1, ob1r, ow2, ob2r, nwr, nbr,
             far, fbr, fcr, fdr, fb1r, fw2r, fb2r, o_ref):
        h = p_ref[0] + p_ref[1]
        dd = d_ref[0] + d_ref[1]
        h = h / dd + b_ref[...]
        h = jnp.where(h > 0, h, jnp.exp(jnp.minimum(h, 0.0)) - 1.0)
        bt = bt_ref[...]
        seg = lax.broadcasted_iota(I32, (B, N), 0)
        oh = (seg == bt.reshape(1, N)).astype(F32)
        sums = jnp.dot(oh, h, preferred_element_type=F32)
        cnt = jnp.sum(oh, axis=1, keepdims=True)
        ce = sums / jnp.maximum(cnt, 1.0)
        obs_e = jnp.dot(
            jnp.maximum(jnp.dot(obs_ref[...], ow1[...],
                                preferred_element_type=F32) + ob1r[...], 0.0),
            ow2[...], preferred_element_type=F32) + ob2r[...]
        noi = jnp.dot(nf_ref[...], nwr[...],
                      preferred_element_type=F32) + nbr[...]
        z = (jnp.dot(ce, far[...], preferred_element_type=F32)
             + jnp.dot(obs_e, fbr[...], preferred_element_type=F32)
             + jnp.dot(noi, fcr[...], preferred_element_type=F32)
             + jnp.dot(ne_ref[...], fdr[...], preferred_element_type=F32)
             + fb1r[...])
        corr = jnp.dot(jnp.maximum(z, 0.0), fw2r[...],
                       preferred_element_type=F32) + fb2r[...]
        o_ref[...] = ne_ref[...] + corr

    args = (p2, den3, b2.reshape(1, 32), bt2, obs, nf, ne,
            oW1, ob1.reshape(1, 32), oW2, ob2.reshape(1, 8),
            nW, nb.reshape(1, 4),
            fa, fb, fc, fd, fb1.reshape(1, 256), fW2, fb2.reshape(1, 1))
    return pl.pallas_call(
        body,
        out_shape=jax.ShapeDtypeStruct((B, 1), F32),
    )(*args)


def kernel(x, edge_index, batch, observable_features, noise_factor,
           noisy_exp, Wl1, Wr1, att1, b1, Wl2, Wr2, att2, b2,
           obs_W1, obs_b1, obs_W2, obs_b2, noise_W, noise_b,
           fus_W1, fus_b1, fus_W2, fus_b2):
    loop = jnp.arange(N, dtype=edge_index.dtype)
    pad = jnp.zeros((EP - E_TOT,), edge_index.dtype)
    src = jnp.concatenate([edge_index[0], loop, pad])
    dst = jnp.concatenate([edge_index[1], loop, pad])

    xl1, xr1 = _tc_dual_matmul(x, Wl1, Wr1, 128, 128)
    p1, d1 = _sc_edge_pass(xl1, xr1, src, dst, att1, 128)
    xl2, xr2 = _tc_norm_elu_matmul(p1, d1, b1, Wl2, Wr2, 128, 32)
    p2, d2 = _sc_edge_pass(xl2, xr2, src, dst, att2, 32)
    obs = observable_features.reshape(B, 5)
    return _tc_final(p2, d2, b2, batch, obs, noise_factor, noisy_exp,
                     obs_W1, obs_b1, obs_W2, obs_b2, noise_W, noise_b,
                     fus_W1, fus_b1, fus_W2, fus_b2)


# speedup vs baseline: 16.0351x; 4.1495x over previous
import functools

import jax
import jax.numpy as jnp
from jax import lax
from jax.experimental import pallas as pl
from jax.experimental.pallas import tpu as pltpu
from jax.experimental.pallas import tpu_sc as plsc

N = 10000
B = 64
E_TOT = 330000
NC, NS, LANES = 2, 16, 16
NW = NC * NS
EP = 331776
PER_W = EP // NW
C = 48
NCH = PER_W // C
NG = C // LANES
ROWS_T = 624

F32 = jnp.float32
I32 = jnp.int32


def _sc_edge_pass(xl, xr, src, dst, att, D):
    mesh = plsc.VectorSubcoreMesh(
        core_axis_name="c", subcore_axis_name="s",
        num_cores=NC, num_subcores=NS)

    @functools.partial(
        pl.kernel,
        out_type=[jax.ShapeDtypeStruct((NC, N, D), F32),
                  jax.ShapeDtypeStruct((NC, N), F32)],
        mesh=mesh,
        compiler_params=pltpu.CompilerParams(
            needs_layout_passes=False,
            use_tc_tiling_on_sc=(D % 128 == 0)),
        scratch_types=[
            pltpu.VMEM_SHARED((N, D), F32),
            pltpu.VMEM_SHARED((N,), F32),
            pltpu.VMEM((2, C), I32),
            pltpu.VMEM((2, C), I32),
            pltpu.VMEM((2, C, D), F32),
            pltpu.VMEM((2, C, D), F32),
            pltpu.VMEM((2, C, D), F32),
            pltpu.VMEM((2, C), F32),
            pltpu.VMEM((2, C), I32),
            pltpu.VMEM((D,), F32),
            pltpu.VMEM((1024,), F32),
            [pltpu.SemaphoreType.DMA] * 2,
            [pltpu.SemaphoreType.DMA] * 2,
            [pltpu.SemaphoreType.DMA] * 2,
        ],
    )
    def k(xl_h, xr_h, src_h, dst_h, att_h, out_h, den_h,
          acc_sh, den_sh, ibs, ibd, bufL, bufR, bufS, exb, sidx, att_v,
          zrow, semG, semS, semI):
        cid = lax.axis_index("c")
        sid = lax.axis_index("s")
        wid = sid * NC + cid
        zero16 = jnp.zeros((LANES,), F32)
        iota = lax.iota(I32, LANES)
        rows = [g * LANES + iota for g in range(NG)]
        cbase = wid * NCH

        def zz(i, c):
            zrow[pl.ds(i * LANES, LANES)] = zero16
            return c
        lax.fori_loop(0, 1024 // LANES, zz, 0)

        def zb(r, c):
            for ko in range(D // LANES):
                bufS[0, r, pl.ds(ko * LANES, LANES)] = zero16
            return c
        lax.fori_loop(0, C, zb, 0)

        r0 = sid * ROWS_T
        for off in range(0, ROWS_T, C):
            r = min(C, ROWS_T - off)
            pltpu.sync_copy(bufS.at[0, pl.ds(0, r)],
                            acc_sh.at[pl.ds(r0 + off, r)])

        @pl.when(sid == NS - 1)
        def _ztail():
            pltpu.sync_copy(bufS.at[0, pl.ds(0, 16)],
                            acc_sh.at[pl.ds(N - 16, 16)])

        @pl.when(sid == 0)
        def _zd():
            for kk in range(N // 1000):
                pltpu.sync_copy(zrow.at[pl.ds(0, 1000)],
                                den_sh.at[pl.ds(kk * 1000, 1000)])

        pltpu.sync_copy(att_h, att_v)
        plsc.subcore_barrier()

        def issue_idx(i, p):
            pltpu.async_copy(src_h.at[cbase + i], ibs.at[p], semI[p])
            pltpu.async_copy(dst_h.at[cbase + i], ibd.at[p], semI[p])

        def wait_idx(p):
            pltpu.make_async_copy(src_h.at[cbase], ibs.at[p],
                                  semI[p]).wait()
            pltpu.make_async_copy(dst_h.at[cbase], ibd.at[p],
                                  semI[p]).wait()

        def issue_rows(p):
            pltpu.async_copy(xl_h.at[ibs.at[p]], bufL.at[p], semG[p])
            pltpu.async_copy(xr_h.at[ibd.at[p]], bufR.at[p], semG[p])

        def wait_rows(p):
            pltpu.make_async_copy(xl_h.at[ibs.at[p]], bufL.at[p],
                                  semG[p]).wait()
            pltpu.make_async_copy(xr_h.at[ibd.at[p]], bufR.at[p],
                                  semG[p]).wait()

        def issue_scat(p):
            pltpu.async_copy(bufS.at[p], acc_sh.at[sidx.at[p]],
                             semS[p], add=True)
            pltpu.async_copy(exb.at[p], den_sh.at[sidx.at[p]],
                             semS[p], add=True)

        def wait_scat(p):
            pltpu.make_async_copy(bufS.at[p], acc_sh.at[sidx.at[p]],
                                  semS[p]).wait()
            pltpu.make_async_copy(exb.at[p], den_sh.at[sidx.at[p]],
                                  semS[p]).wait()

        def do_chunk(i, p, wait_i, wait_s):
            wait_rows(p)
            if wait_s:
                wait_scat(p)
            for g in range(NG):
                sidx[p, pl.ds(g * LANES, LANES)] = (
                    ibd[p, pl.ds(g * LANES, LANES)])

            @pl.when(i + 2 < NCH)
            def _():
                issue_idx(i + 2, p)
            if wait_i:
                @pl.when(i + 1 < NCH)
                def _():
                    wait_idx(1 - p)
                    issue_rows(1 - p)
            else:
                issue_rows(1 - p)

            def fA(f, accs):
                rot = (f + iota) & (D - 1)
                av = plsc.load_gather(att_v, [rot])
                out = []
                for g in range(NG):
                    a = plsc.load_gather(bufL.at[p], [rows[g], rot])
                    b = plsc.load_gather(bufR.at[p], [rows[g], rot])
                    h = a + b
                    h = jnp.where(h > 0, h, 0.2 * h)
                    out.append(accs[g] + h * av)
                return tuple(out)
            accs = lax.fori_loop(0, D, fA,
                                 tuple(zero16 for _ in range(NG)),
                                 unroll=4)

            base_e = (cbase + i) * C
            for g in range(NG):
                gid = base_e + rows[g]
                ex = jnp.where(gid < E_TOT, jnp.exp(accs[g]), 0.0)
                exb[p, pl.ds(g * LANES, LANES)] = ex

            exs = [exb[p, pl.ds(g * LANES, LANES)] for g in range(NG)]

            def fB(f, c):
                rot = (f + iota) & (D - 1)
                for g in range(NG):
                    v = plsc.load_gather(bufL.at[p], [rows[g], rot])
                    plsc.store_scatter(bufS.at[p], [rows[g], rot],
                                       v * exs[g])
                return c
            lax.fori_loop(0, D, fB, 0, unroll=4)
            issue_scat(p)

        for p in (0, 1):
            issue_idx(p, p)
            wait_idx(p)
        issue_rows(0)
        do_chunk(0, 0, wait_i=False, wait_s=False)
        do_chunk(1, 1, wait_i=True, wait_s=False)

        def pair(j, c):
            do_chunk(2 * j, 0, wait_i=True, wait_s=True)
            do_chunk(2 * j + 1, 1, wait_i=True, wait_s=True)
            return c
        lax.fori_loop(1, NCH // 2, pair, 0)

        wait_scat(0)
        wait_scat(1)

        plsc.subcore_barrier()

        pltpu.sync_copy(acc_sh.at[pl.ds(r0, ROWS_T)],
                        out_h.at[cid, pl.ds(r0, ROWS_T)])

        @pl.when(sid == NS - 1)
        def _wtail():
            pltpu.sync_copy(acc_sh.at[pl.ds(N - 16, 16)],
                            out_h.at[cid, pl.ds(N - 16, 16)])

        @pl.when(sid == 0)
        def _wd():
            pltpu.sync_copy(den_sh, den_h.at[cid])

    return k(xl, xr, src.reshape(EP // C, C), dst.reshape(EP // C, C),
             att)


def _tc_dual_matmul(x, Wl, Wr, K, M):
    R = 400

    def body(x_ref, wl_ref, wr_ref, ol_ref, or_ref):
        xb = x_ref[...]
        ol_ref[...] = jnp.dot(xb, wl_ref[...], preferred_element_type=F32)
        or_ref[...] = jnp.dot(xb, wr_ref[...], preferred_element_type=F32)

    return pl.pallas_call(
        body,
        grid=(N // R,),
        in_specs=[pl.BlockSpec((R, K), lambda i: (i, 0)),
                  pl.BlockSpec((K, M), lambda i: (0, 0)),
                  pl.BlockSpec((K, M), lambda i: (0, 0))],
        out_specs=[pl.BlockSpec((R, M), lambda i: (i, 0)),
                   pl.BlockSpec((R, M), lambda i: (i, 0))],
        out_shape=[jax.ShapeDtypeStruct((N, M), F32),
                   jax.ShapeDtypeStruct((N, M), F32)],
    )(x, Wl, Wr)


def _tc_norm_elu_matmul(p, den, bias, Wl, Wr, K, M):
    R = 400
    den3 = den.reshape(NC, N, 1)
    b2 = bias.reshape(1, K)

    def body(p_ref, d_ref, b_ref, wl_ref, wr_ref, ol_ref, or_ref):
        h = p_ref[0] + p_ref[1]
        dd = d_ref[0] + d_ref[1]
        h = h / dd + b_ref[...]
        h = jnp.where(h > 0, h, jnp.exp(jnp.minimum(h, 0.0)) - 1.0)
        ol_ref[...] = jnp.dot(h, wl_ref[...], preferred_element_type=F32)
        or_ref[...] = jnp.dot(h, wr_ref[...], preferred_element_type=F32)

    return pl.pallas_call(
        body,
        grid=(N // R,),
        in_specs=[pl.BlockSpec((NC, R, K), lambda i: (0, i, 0)),
                  pl.BlockSpec((NC, R, 1), lambda i: (0, i, 0)),
                  pl.BlockSpec((1, K), lambda i: (0, 0)),
                  pl.BlockSpec((K, M), lambda i: (0, 0)),
                  pl.BlockSpec((K, M), lambda i: (0, 0))],
        out_specs=[pl.BlockSpec((R, M), lambda i: (i, 0)),
                   pl.BlockSpec((R, M), lambda i: (i, 0))],
        out_shape=[jax.ShapeDtypeStruct((N, M), F32),
                   jax.ShapeDtypeStruct((N, M), F32)],
    )(p, den3, b2, Wl, Wr)


def _tc_final(p2, den2, b2, batch, obs, nf, ne,
              oW1, ob1, oW2, ob2, nW, nb, fW1, fb1, fW2, fb2):
    den3 = den2.reshape(NC, N, 1)
    bt2 = batch.reshape(N, 1)
    fa, fb, fc, fd = fW1[:32], fW1[32:40], fW1[40:44], fW1[44:45]

    def body(p_ref, d_ref, b_ref, bt_ref, obs_ref, nf_ref, ne_ref,
             ow1, ob1r, ow2, ob2r, nwr, nbr,
             far, fbr, fcr, fdr, fb1r, fw2r, fb2r, o_ref):
        h = p_ref[0] + p_ref[1]
        dd = d_ref[0] + d_ref[1]
        h = h / dd + b_ref[...]
        h = jnp.where(h > 0, h, jnp.exp(jnp.minimum(h, 0.0)) - 1.0)
        bt = bt_ref[...]
        seg = lax.broadcasted_iota(I32, (B, N), 0)
        oh = (seg == bt.reshape(1, N)).astype(F32)
        sums = jnp.dot(oh, h, preferred_element_type=F32)
        cnt = jnp.sum(oh, axis=1, keepdims=True)
        ce = sums / jnp.maximum(cnt, 1.0)
        obs_e = jnp.dot(
            jnp.maximum(jnp.dot(obs_ref[...], ow1[...],
                                preferred_element_type=F32) + ob1r[...], 0.0),
            ow2[...], preferred_element_type=F32) + ob2r[...]
        noi = jnp.dot(nf_ref[...], nwr[...],
                      preferred_element_type=F32) + nbr[...]
        z = (jnp.dot(ce, far[...], preferred_element_type=F32)
             + jnp.dot(obs_e, fbr[...], preferred_element_type=F32)
             + jnp.dot(noi, fcr[...], preferred_element_type=F32)
             + jnp.dot(ne_ref[...], fdr[...], preferred_element_type=F32)
             + fb1r[...])
        corr = jnp.dot(jnp.maximum(z, 0.0), fw2r[...],
                       preferred_element_type=F32) + fb2r[...]
        o_ref[...] = ne_ref[...] + corr

    args = (p2, den3, b2.reshape(1, 32), bt2, obs, nf, ne,
            oW1, ob1.reshape(1, 32), oW2, ob2.reshape(1, 8),
            nW, nb.reshape(1, 4),
            fa, fb, fc, fd, fb1.reshape(1, 256), fW2, fb2.reshape(1, 1))
    return pl.pallas_call(
        body,
        out_shape=jax.ShapeDtypeStruct((B, 1), F32),
    )(*args)


def kernel(x, edge_index, batch, observable_features, noise_factor,
           noisy_exp, Wl1, Wr1, att1, b1, Wl2, Wr2, att2, b2,
           obs_W1, obs_b1, obs_W2, obs_b2, noise_W, noise_b,
           fus_W1, fus_b1, fus_W2, fus_b2):
    loop = jnp.arange(N, dtype=edge_index.dtype)
    pad = jnp.zeros((EP - E_TOT,), edge_index.dtype)
    src = jnp.concatenate([edge_index[0], loop, pad])
    dst = jnp.concatenate([edge_index[1], loop, pad])

    xl1, xr1 = _tc_dual_matmul(x, Wl1, Wr1, 128, 128)
    p1, d1 = _sc_edge_pass(xl1, xr1, src, dst, att1, 128)
    xl2, xr2 = _tc_norm_elu_matmul(p1, d1, b1, Wl2, Wr2, 128, 32)
    p2, d2 = _sc_edge_pass(xl2, xr2, src, dst, att2, 32)
    obs = observable_features.reshape(B, 5)
    return _tc_final(p2, d2, b2, batch, obs, noise_factor, noisy_exp,
                     obs_W1, obs_b1, obs_W2, obs_b2, noise_W, noise_b,
                     fus_W1, fus_b1, fus_W2, fus_b2)


# speedup vs baseline: 16.0905x; 1.0035x over previous
import functools

import jax
import jax.numpy as jnp
from jax import lax
from jax.experimental import pallas as pl
from jax.experimental.pallas import tpu as pltpu
from jax.experimental.pallas import tpu_sc as plsc

N = 10000
B = 64
E_TOT = 330000
NC, NS, LANES = 2, 16, 16
NW = NC * NS
EP = 331776
PER_W = EP // NW
C = 64
NCH = PER_W // C
NG = C // LANES
ROWS_T = 624

F32 = jnp.float32
I32 = jnp.int32


def _sc_edge_pass(xl, xr, src, dst, att, D):
    mesh = plsc.VectorSubcoreMesh(
        core_axis_name="c", subcore_axis_name="s",
        num_cores=NC, num_subcores=NS)

    @functools.partial(
        pl.kernel,
        out_type=[jax.ShapeDtypeStruct((NC, N, D), F32),
                  jax.ShapeDtypeStruct((NC, N), F32)],
        mesh=mesh,
        compiler_params=pltpu.CompilerParams(
            needs_layout_passes=False,
            use_tc_tiling_on_sc=(D % 128 == 0)),
        scratch_types=[
            pltpu.VMEM_SHARED((N, D), F32),
            pltpu.VMEM_SHARED((N,), F32),
            pltpu.VMEM((2, C), I32),
            pltpu.VMEM((2, C), I32),
            pltpu.VMEM((2, C, D), F32),
            pltpu.VMEM((2, C, D), F32),
            pltpu.VMEM((2, C, D), F32),
            pltpu.VMEM((2, C), F32),
            pltpu.VMEM((2, C), I32),
            pltpu.VMEM((D,), F32),
            [pltpu.SemaphoreType.DMA] * 2,
            [pltpu.SemaphoreType.DMA] * 2,
            [pltpu.SemaphoreType.DMA] * 2,
        ],
    )
    def k(xl_h, xr_h, src_h, dst_h, att_h, out_h, den_h,
          acc_sh, den_sh, ibs, ibd, bufL, bufR, bufS, exb, sidx, att_v,
          semG, semS, semI):
        cid = lax.axis_index("c")
        sid = lax.axis_index("s")
        wid = sid * NC + cid
        zero16 = jnp.zeros((LANES,), F32)
        iota = lax.iota(I32, LANES)
        rows = [g * LANES + iota for g in range(NG)]
        cbase = wid * NCH

        for g in range(NG):
            exb[0, pl.ds(g * LANES, LANES)] = zero16

        def zb(r, c):
            for ko in range(D // LANES):
                bufS[0, r, pl.ds(ko * LANES, LANES)] = zero16
            return c
        lax.fori_loop(0, C, zb, 0)

        r0 = sid * ROWS_T
        for off in range(0, ROWS_T, C):
            r = min(C, ROWS_T - off)
            pltpu.sync_copy(bufS.at[0, pl.ds(0, r)],
                            acc_sh.at[pl.ds(r0 + off, r)])

        @pl.when(sid == NS - 1)
        def _ztail():
            pltpu.sync_copy(bufS.at[0, pl.ds(0, 16)],
                            acc_sh.at[pl.ds(N - 16, 16)])

        d0 = sid * 640
        for kk in range(10):
            off = kk * 64
            if kk < 6:
                pltpu.sync_copy(exb.at[0], den_sh.at[pl.ds(d0 + off, 64)])
            else:
                @pl.when(sid < NS - 1)
                def _zm(off=off):
                    pltpu.sync_copy(exb.at[0],
                                    den_sh.at[pl.ds(d0 + off, 64)])
        @pl.when(sid == NS - 1)
        def _zt():
            pltpu.sync_copy(exb.at[0, pl.ds(0, 16)],
                            den_sh.at[pl.ds(N - 16, 16)])

        pltpu.sync_copy(att_h, att_v)
        plsc.subcore_barrier()

        def issue_idx(i, p):
            pltpu.async_copy(src_h.at[cbase + i], ibs.at[p], semI[p])
            pltpu.async_copy(dst_h.at[cbase + i], ibd.at[p], semI[p])

        def wait_idx(p):
            pltpu.make_async_copy(src_h.at[cbase], ibs.at[p],
                                  semI[p]).wait()
            pltpu.make_async_copy(dst_h.at[cbase], ibd.at[p],
                                  semI[p]).wait()

        def issue_rows(p):
            pltpu.async_copy(xl_h.at[ibs.at[p]], bufL.at[p], semG[p])
            pltpu.async_copy(xr_h.at[ibd.at[p]], bufR.at[p], semG[p])

        def wait_rows(p):
            pltpu.make_async_copy(xl_h.at[ibs.at[p]], bufL.at[p],
                                  semG[p]).wait()
            pltpu.make_async_copy(xr_h.at[ibd.at[p]], bufR.at[p],
                                  semG[p]).wait()

        def issue_scat(p):
            pltpu.async_copy(bufS.at[p], acc_sh.at[sidx.at[p]],
                             semS[p], add=True)
            pltpu.async_copy(exb.at[p], den_sh.at[sidx.at[p]],
                             semS[p], add=True)

        def wait_scat(p):
            pltpu.make_async_copy(bufS.at[p], acc_sh.at[sidx.at[p]],
                                  semS[p]).wait()
            pltpu.make_async_copy(exb.at[p], den_sh.at[sidx.at[p]],
                                  semS[p]).wait()

        def do_chunk(i, p, wait_i, wait_s):
            wait_rows(p)
            if wait_s:
                wait_scat(p)
            for g in range(NG):
                sidx[p, pl.ds(g * LANES, LANES)] = (
                    ibd[p, pl.ds(g * LANES, LANES)])

            @pl.when(i + 2 < NCH)
            def _():
                issue_idx(i + 2, p)
            if wait_i:
                @pl.when(i + 1 < NCH)
                def _():
                    wait_idx(1 - p)
                    issue_rows(1 - p)
            else:
                issue_rows(1 - p)

            def fA(f, accs):
                rot = (f + iota) & (D - 1)
                av = plsc.load_gather(att_v, [rot])
                out = []
                for g in range(NG):
                    a = plsc.load_gather(bufL.at[p], [rows[g], rot])
                    b = plsc.load_gather(bufR.at[p], [rows[g], rot])
                    h = a + b
                    h = jnp.where(h > 0, h, 0.2 * h)
                    out.append(accs[g] + h * av)
                return tuple(out)
            accs = lax.fori_loop(0, D, fA,
                                 tuple(zero16 for _ in range(NG)),
                                 unroll=4)

            base_e = (cbase + i) * C
            for g in range(NG):
                gid = base_e + rows[g]
                ex = jnp.where(gid < E_TOT, jnp.exp(accs[g]), 0.0)
                exb[p, pl.ds(g * LANES, LANES)] = ex

            exs = [exb[p, pl.ds(g * LANES, LANES)] for g in range(NG)]

            def fB(f, c):
                rot = (f + iota) & (D - 1)
                for g in range(NG):
                    v = plsc.load_gather(bufL.at[p], [rows[g], rot])
                    plsc.store_scatter(bufS.at[p], [rows[g], rot],
                                       v * exs[g])
                return c
            lax.fori_loop(0, D, fB, 0, unroll=4)
            issue_scat(p)

        for p in (0, 1):
            issue_idx(p, p)
            wait_idx(p)
        issue_rows(0)
        do_chunk(0, 0, wait_i=False, wait_s=False)
        do_chunk(1, 1, wait_i=True, wait_s=False)

        def pair(j, c):
            do_chunk(2 * j, 0, wait_i=True, wait_s=True)
            do_chunk(2 * j + 1, 1, wait_i=True, wait_s=True)
            return c
        lax.fori_loop(1, NCH // 2, pair, 0)

        wait_scat(0)
        wait_scat(1)

        plsc.subcore_barrier()

        pltpu.sync_copy(acc_sh.at[pl.ds(r0, ROWS_T)],
                        out_h.at[cid, pl.ds(r0, ROWS_T)])

        @pl.when(sid == NS - 1)
        def _wtail():
            pltpu.sync_copy(acc_sh.at[pl.ds(N - 16, 16)],
                            out_h.at[cid, pl.ds(N - 16, 16)])

        @pl.when(sid == 0)
        def _wd():
            pltpu.sync_copy(den_sh, den_h.at[cid])

    return k(xl, xr, src.reshape(EP // C, C), dst.reshape(EP // C, C),
             att)


def _tc_dual_matmul(x, Wl, Wr, K, M):
    R = 400

    def body(x_ref, wl_ref, wr_ref, ol_ref, or_ref):
        xb = x_ref[...]
        ol_ref[...] = jnp.dot(xb, wl_ref[...], preferred_element_type=F32)
        or_ref[...] = jnp.dot(xb, wr_ref[...], preferred_element_type=F32)

    return pl.pallas_call(
        body,
        grid=(N // R,),
        in_specs=[pl.BlockSpec((R, K), lambda i: (i, 0)),
                  pl.BlockSpec((K, M), lambda i: (0, 0)),
                  pl.BlockSpec((K, M), lambda i: (0, 0))],
        out_specs=[pl.BlockSpec((R, M), lambda i: (i, 0)),
                   pl.BlockSpec((R, M), lambda i: (i, 0))],
        out_shape=[jax.ShapeDtypeStruct((N, M), F32),
                   jax.ShapeDtypeStruct((N, M), F32)],
    )(x, Wl, Wr)


def _tc_norm_elu_matmul(p, den, bias, Wl, Wr, K, M):
    R = 400
    den3 = den.reshape(NC, N, 1)
    b2 = bias.reshape(1, K)

    def body(p_ref, d_ref, b_ref, wl_ref, wr_ref, ol_ref, or_ref):
        h = p_ref[0] + p_ref[1]
        dd = d_ref[0] + d_ref[1]
        h = h / dd + b_ref[...]
        h = jnp.where(h > 0, h, jnp.exp(jnp.minimum(h, 0.0)) - 1.0)
        ol_ref[...] = jnp.dot(h, wl_ref[...], preferred_element_type=F32)
        or_ref[...] = jnp.dot(h, wr_ref[...], preferred_element_type=F32)

    return pl.pallas_call(
        body,
        grid=(N // R,),
        in_specs=[pl.BlockSpec((NC, R, K), lambda i: (0, i, 0)),
                  pl.BlockSpec((NC, R, 1), lambda i: (0, i, 0)),
                  pl.BlockSpec((1, K), lambda i: (0, 0)),
                  pl.BlockSpec((K, M), lambda i: (0, 0)),
                  pl.BlockSpec((K, M), lambda i: (0, 0))],
        out_specs=[pl.BlockSpec((R, M), lambda i: (i, 0)),
                   pl.BlockSpec((R, M), lambda i: (i, 0))],
        out_shape=[jax.ShapeDtypeStruct((N, M), F32),
                   jax.ShapeDtypeStruct((N, M), F32)],
    )(p, den3, b2, Wl, Wr)


def _tc_final(p2, den2, b2, batch, obs, nf, ne,
              oW1, ob1, oW2, ob2, nW, nb, fW1, fb1, fW2, fb2):
    den3 = den2.reshape(NC, N, 1)
    bt2 = batch.reshape(N, 1)
    fa, fb, fc, fd = fW1[:32], fW1[32:40], fW1[40:44], fW1[44:45]

    def body(p_ref, d_ref, b_ref, bt_ref, obs_ref, nf_ref, ne_ref,
             ow1, ob1r, ow2, ob2r, nwr, nbr,
             far, fbr, fcr, fdr, fb1r, fw2r, fb2r, o_ref):
        h = p_ref[0] + p_ref[1]
        dd = d_ref[0] + d_ref[1]
        h = h / dd + b_ref[...]
        h = jnp.where(h > 0, h, jnp.exp(jnp.minimum(h, 0.0)) - 1.0)
        bt = bt_ref[...]
        seg = lax.broadcasted_iota(I32, (B, N), 0)
        oh = (seg == bt.reshape(1, N)).astype(F32)
        sums = jnp.dot(oh, h, preferred_element_type=F32)
        cnt = jnp.sum(oh, axis=1, keepdims=True)
        ce = sums / jnp.maximum(cnt, 1.0)
        obs_e = jnp.dot(
            jnp.maximum(jnp.dot(obs_ref[...], ow1[...],
                                preferred_element_type=F32) + ob1r[...], 0.0),
            ow2[...], preferred_element_type=F32) + ob2r[...]
        noi = jnp.dot(nf_ref[...], nwr[...],
                      preferred_element_type=F32) + nbr[...]
        z = (jnp.dot(ce, far[...], preferred_element_type=F32)
             + jnp.dot(obs_e, fbr[...], preferred_element_type=F32)
             + jnp.dot(noi, fcr[...], preferred_element_type=F32)
             + jnp.dot(ne_ref[...], fdr[...], preferred_element_type=F32)
             + fb1r[...])
        corr = jnp.dot(jnp.maximum(z, 0.0), fw2r[...],
                       preferred_element_type=F32) + fb2r[...]
        o_ref[...] = ne_ref[...] + corr

    args = (p2, den3, b2.reshape(1, 32), bt2, obs, nf, ne,
            oW1, ob1.reshape(1, 32), oW2, ob2.reshape(1, 8),
            nW, nb.reshape(1, 4),
            fa, fb, fc, fd, fb1.reshape(1, 256), fW2, fb2.reshape(1, 1))
    return pl.pallas_call(
        body,
        out_shape=jax.ShapeDtypeStruct((B, 1), F32),
    )(*args)


def kernel(x, edge_index, batch, observable_features, noise_factor,
           noisy_exp, Wl1, Wr1, att1, b1, Wl2, Wr2, att2, b2,
           obs_W1, obs_b1, obs_W2, obs_b2, noise_W, noise_b,
           fus_W1, fus_b1, fus_W2, fus_b2):
    loop = jnp.arange(N, dtype=edge_index.dtype)
    pad = jnp.zeros((EP - E_TOT,), edge_index.dtype)
    src = jnp.concatenate([edge_index[0], loop, pad])
    dst = jnp.concatenate([edge_index[1], loop, pad])

    xl1, xr1 = _tc_dual_matmul(x, Wl1, Wr1, 128, 128)
    p1, d1 = _sc_edge_pass(xl1, xr1, src, dst, att1, 128)
    xl2, xr2 = _tc_norm_elu_matmul(p1, d1, b1, Wl2, Wr2, 128, 32)
    p2, d2 = _sc_edge_pass(xl2, xr2, src, dst, att2, 32)
    obs = observable_features.reshape(B, 5)
    return _tc_final(p2, d2, b2, batch, obs, noise_factor, noisy_exp,
                     obs_W1, obs_b1, obs_W2, obs_b2, noise_W, noise_b,
                     fus_W1, fus_b1, fus_W2, fus_b2)


# speedup vs baseline: 18.3946x; 1.1432x over previous
import functools

import jax
import jax.numpy as jnp
from jax import lax
from jax.experimental import pallas as pl
from jax.experimental.pallas import tpu as pltpu
from jax.experimental.pallas import tpu_sc as plsc

N = 10000
B = 64
E_TOT = 330000
NC, NS, LANES = 2, 16, 16
NW = NC * NS
EP = 331776
PER_W = EP // NW
C = 64
NCH = PER_W // C
NG = C // LANES
ROWS_T = 624

F32 = jnp.float32
I32 = jnp.int32


def _pack2(x):
    h = x.astype(jnp.bfloat16)
    return lax.bitcast_convert_type(
        h.reshape(*h.shape[:-1], h.shape[-1] // 2, 2), I32)


def _sc_edge_pass(xl, xr, src, dst, att, D):
    W = D // 2
    mesh = plsc.VectorSubcoreMesh(
        core_axis_name="c", subcore_axis_name="s",
        num_cores=NC, num_subcores=NS)

    @functools.partial(
        pl.kernel,
        out_type=[jax.ShapeDtypeStruct((NC, N, D), F32),
                  jax.ShapeDtypeStruct((NC, N), F32)],
        mesh=mesh,
        compiler_params=pltpu.CompilerParams(
            needs_layout_passes=False,
            use_tc_tiling_on_sc=False),
        scratch_types=[
            pltpu.VMEM_SHARED((N, D), F32),
            pltpu.VMEM_SHARED((N,), F32),
            pltpu.VMEM((2, C), I32),
            pltpu.VMEM((2, C), I32),
            pltpu.VMEM((2, C, W), I32),
            pltpu.VMEM((2, C, W), I32),
            pltpu.VMEM((2, C, D), F32),
            pltpu.VMEM((2, C), F32),
            pltpu.VMEM((2, C), I32),
            pltpu.VMEM((W,), I32),
            [pltpu.SemaphoreType.DMA] * 2,
            [pltpu.SemaphoreType.DMA] * 2,
            [pltpu.SemaphoreType.DMA] * 2,
        ],
    )
    def k(xl_h, xr_h, src_h, dst_h, att_h, out_h, den_h,
          acc_sh, den_sh, ibs, ibd, bufL, bufR, bufS, exb, sidx, att_v,
          semG, semS, semI):
        cid = lax.axis_index("c")
        sid = lax.axis_index("s")
        wid = sid * NC + cid
        zero16 = jnp.zeros((LANES,), F32)
        iota = lax.iota(I32, LANES)
        rows = [g * LANES + iota for g in range(NG)]
        cbase = wid * NCH

        for g in range(NG):
            exb[0, pl.ds(g * LANES, LANES)] = zero16

        def zb(r, c):
            for ko in range(D // LANES):
                bufS[0, r, pl.ds(ko * LANES, LANES)] = zero16
            return c
        lax.fori_loop(0, C, zb, 0)

        r0 = sid * ROWS_T
        for off in range(0, ROWS_T, C):
            r = min(C, ROWS_T - off)
            pltpu.sync_copy(bufS.at[0, pl.ds(0, r)],
                            acc_sh.at[pl.ds(r0 + off, r)])

        @pl.when(sid == NS - 1)
        def _ztail():
            pltpu.sync_copy(bufS.at[0, pl.ds(0, 16)],
                            acc_sh.at[pl.ds(N - 16, 16)])

        d0 = sid * 640
        for kk in range(10):
            off = kk * 64
            if kk < 6:
                pltpu.sync_copy(exb.at[0], den_sh.at[pl.ds(d0 + off, 64)])
            else:
                @pl.when(sid < NS - 1)
                def _zm(off=off):
                    pltpu.sync_copy(exb.at[0],
                                    den_sh.at[pl.ds(d0 + off, 64)])
        @pl.when(sid == NS - 1)
        def _zt():
            pltpu.sync_copy(exb.at[0, pl.ds(0, 16)],
                            den_sh.at[pl.ds(N - 16, 16)])

        pltpu.sync_copy(att_h, att_v)
        plsc.subcore_barrier()

        def issue_idx(i, p):
            pltpu.async_copy(src_h.at[cbase + i], ibs.at[p], semI[p])
            pltpu.async_copy(dst_h.at[cbase + i], ibd.at[p], semI[p])

        def wait_idx(p):
            pltpu.make_async_copy(src_h.at[cbase], ibs.at[p],
                                  semI[p]).wait()
            pltpu.make_async_copy(dst_h.at[cbase], ibd.at[p],
                                  semI[p]).wait()

        def issue_rows(p):
            pltpu.async_copy(xl_h.at[ibs.at[p]], bufL.at[p], semG[p])
            pltpu.async_copy(xr_h.at[ibd.at[p]], bufR.at[p], semG[p])

        def wait_rows(p):
            pltpu.make_async_copy(xl_h.at[ibs.at[p]], bufL.at[p],
                                  semG[p]).wait()
            pltpu.make_async_copy(xr_h.at[ibd.at[p]], bufR.at[p],
                                  semG[p]).wait()

        def issue_scat(p):
            pltpu.async_copy(bufS.at[p], acc_sh.at[sidx.at[p]],
                             semS[p], add=True)
            pltpu.async_copy(exb.at[p], den_sh.at[sidx.at[p]],
                             semS[p], add=True)

        def wait_scat(p):
            pltpu.make_async_copy(bufS.at[p], acc_sh.at[sidx.at[p]],
                                  semS[p]).wait()
            pltpu.make_async_copy(exb.at[p], den_sh.at[sidx.at[p]],
                                  semS[p]).wait()

        def do_chunk(i, p, wait_i, wait_s):
            wait_rows(p)
            if wait_s:
                wait_scat(p)
            for g in range(NG):
                sidx[p, pl.ds(g * LANES, LANES)] = (
                    ibd[p, pl.ds(g * LANES, LANES)])

            @pl.when(i + 2 < NCH)
            def _():
                issue_idx(i + 2, p)
            if wait_i:
                @pl.when(i + 1 < NCH)
                def _():
                    wait_idx(1 - p)
                    issue_rows(1 - p)
            else:
                issue_rows(1 - p)

            def _up(x):
                return plsc.unpack(
                    plsc.bitcast(x, jnp.bfloat16),
                    format=plsc.PackFormat.INTERLEAVED,
                    preferred_element_type=F32)

            def fA(w, accs):
                rot = (w + iota) & (W - 1)
                a0, a1 = _up(plsc.load_gather(att_v, [rot]))
                out = []
                for g in range(NG):
                    l0, l1 = _up(plsc.load_gather(bufL.at[p],
                                                  [rows[g], rot]))
                    r0, r1 = _up(plsc.load_gather(bufR.at[p],
                                                  [rows[g], rot]))
                    h0 = l0 + r0
                    h1 = l1 + r1
                    h0 = jnp.where(h0 > 0, h0, 0.2 * h0)
                    h1 = jnp.where(h1 > 0, h1, 0.2 * h1)
                    out.append(accs[g] + h0 * a0 + h1 * a1)
                return tuple(out)
            accs = lax.fori_loop(0, W, fA,
                                 tuple(zero16 for _ in range(NG)),
                                 unroll=4)

            base_e = (cbase + i) * C
            for g in range(NG):
                gid = base_e + rows[g]
                ex = jnp.where(gid < E_TOT, jnp.exp(accs[g]), 0.0)
                exb[p, pl.ds(g * LANES, LANES)] = ex

            exs = [exb[p, pl.ds(g * LANES, LANES)] for g in range(NG)]

            def fB(w, c):
                rot = (w + iota) & (W - 1)
                f0 = rot * 2
                for g in range(NG):
                    v0, v1 = _up(plsc.load_gather(bufL.at[p],
                                                  [rows[g], rot]))
                    plsc.store_scatter(bufS.at[p], [rows[g], f0],
                                       v0 * exs[g])
                    plsc.store_scatter(bufS.at[p], [rows[g], f0 + 1],
                                       v1 * exs[g])
                return c
            lax.fori_loop(0, W, fB, 0, unroll=4)
            issue_scat(p)

        for p in (0, 1):
            issue_idx(p, p)
            wait_idx(p)
        issue_rows(0)
        do_chunk(0, 0, wait_i=False, wait_s=False)
        do_chunk(1, 1, wait_i=True, wait_s=False)

        def pair(j, c):
            do_chunk(2 * j, 0, wait_i=True, wait_s=True)
            do_chunk(2 * j + 1, 1, wait_i=True, wait_s=True)
            return c
        lax.fori_loop(1, NCH // 2, pair, 0)

        wait_scat(0)
        wait_scat(1)

        plsc.subcore_barrier()

        pltpu.sync_copy(acc_sh.at[pl.ds(r0, ROWS_T)],
                        out_h.at[cid, pl.ds(r0, ROWS_T)])

        @pl.when(sid == NS - 1)
        def _wtail():
            pltpu.sync_copy(acc_sh.at[pl.ds(N - 16, 16)],
                            out_h.at[cid, pl.ds(N - 16, 16)])

        @pl.when(sid == 0)
        def _wd():
            pltpu.sync_copy(den_sh, den_h.at[cid])

    return k(xl, xr, src.reshape(EP // C, C), dst.reshape(EP // C, C),
             att)


def _tc_dual_matmul(x, Wl, Wr, K, M):
    R = 400

    def body(x_ref, wl_ref, wr_ref, ol_ref, or_ref):
        xb = x_ref[...]
        ol_ref[...] = jnp.dot(xb, wl_ref[...], preferred_element_type=F32)
        or_ref[...] = jnp.dot(xb, wr_ref[...], preferred_element_type=F32)

    return pl.pallas_call(
        body,
        grid=(N // R,),
        in_specs=[pl.BlockSpec((R, K), lambda i: (i, 0)),
                  pl.BlockSpec((K, M), lambda i: (0, 0)),
                  pl.BlockSpec((K, M), lambda i: (0, 0))],
        out_specs=[pl.BlockSpec((R, M), lambda i: (i, 0)),
                   pl.BlockSpec((R, M), lambda i: (i, 0))],
        out_shape=[jax.ShapeDtypeStruct((N, M), F32),
                   jax.ShapeDtypeStruct((N, M), F32)],
    )(x, Wl, Wr)


def _tc_norm_elu_matmul(p, den, bias, Wl, Wr, K, M):
    R = 400
    den3 = den.reshape(NC, N, 1)
    b2 = bias.reshape(1, K)

    def body(p_ref, d_ref, b_ref, wl_ref, wr_ref, ol_ref, or_ref):
        h = p_ref[0] + p_ref[1]
        dd = d_ref[0] + d_ref[1]
        h = h / dd + b_ref[...]
        h = jnp.where(h > 0, h, jnp.exp(jnp.minimum(h, 0.0)) - 1.0)
        ol_ref[...] = jnp.dot(h, wl_ref[...], preferred_element_type=F32)
        or_ref[...] = jnp.dot(h, wr_ref[...], preferred_element_type=F32)

    return pl.pallas_call(
        body,
        grid=(N // R,),
        in_specs=[pl.BlockSpec((NC, R, K), lambda i: (0, i, 0)),
                  pl.BlockSpec((NC, R, 1), lambda i: (0, i, 0)),
                  pl.BlockSpec((1, K), lambda i: (0, 0)),
                  pl.BlockSpec((K, M), lambda i: (0, 0)),
                  pl.BlockSpec((K, M), lambda i: (0, 0))],
        out_specs=[pl.BlockSpec((R, M), lambda i: (i, 0)),
                   pl.BlockSpec((R, M), lambda i: (i, 0))],
        out_shape=[jax.ShapeDtypeStruct((N, M), F32),
                   jax.ShapeDtypeStruct((N, M), F32)],
    )(p, den3, b2, Wl, Wr)


def _tc_final(p2, den2, b2, batch, obs, nf, ne,
              oW1, ob1, oW2, ob2, nW, nb, fW1, fb1, fW2, fb2):
    den3 = den2.reshape(NC, N, 1)
    bt2 = batch.reshape(N, 1)
    fa, fb, fc, fd = fW1[:32], fW1[32:40], fW1[40:44], fW1[44:45]

    def body(p_ref, d_ref, b_ref, bt_ref, obs_ref, nf_ref, ne_ref,
             ow1, ob1r, ow2, ob2r, nwr, nbr,
             far, fbr, fcr, fdr, fb1r, fw2r, fb2r, o_ref):
        h = p_ref[0] + p_ref[1]
        dd = d_ref[0] + d_ref[1]
        h = h / dd + b_ref[...]
        h = jnp.where(h > 0, h, jnp.exp(jnp.minimum(h, 0.0)) - 1.0)
        bt = bt_ref[...]
        seg = lax.broadcasted_iota(I32, (B, N), 0)
        oh = (seg == bt.reshape(1, N)).astype(F32)
        sums = jnp.dot(oh, h, preferred_element_type=F32)
        cnt = jnp.sum(oh, axis=1, keepdims=True)
        ce = sums / jnp.maximum(cnt, 1.0)
        obs_e = jnp.dot(
            jnp.maximum(jnp.dot(obs_ref[...], ow1[...],
                                preferred_element_type=F32) + ob1r[...], 0.0),
            ow2[...], preferred_element_type=F32) + ob2r[...]
        noi = jnp.dot(nf_ref[...], nwr[...],
                      preferred_element_type=F32) + nbr[...]
        z = (jnp.dot(ce, far[...], preferred_element_type=F32)
             + jnp.dot(obs_e, fbr[...], preferred_element_type=F32)
             + jnp.dot(noi, fcr[...], preferred_element_type=F32)
             + jnp.dot(ne_ref[...], fdr[...], preferred_element_type=F32)
             + fb1r[...])
        corr = jnp.dot(jnp.maximum(z, 0.0), fw2r[...],
                       preferred_element_type=F32) + fb2r[...]
        o_ref[...] = ne_ref[...] + corr

    args = (p2, den3, b2.reshape(1, 32), bt2, obs, nf, ne,
            oW1, ob1.reshape(1, 32), oW2, ob2.reshape(1, 8),
            nW, nb.reshape(1, 4),
            fa, fb, fc, fd, fb1.reshape(1, 256), fW2, fb2.reshape(1, 1))
    return pl.pallas_call(
        body,
        out_shape=jax.ShapeDtypeStruct((B, 1), F32),
    )(*args)


def kernel(x, edge_index, batch, observable_features, noise_factor,
           noisy_exp, Wl1, Wr1, att1, b1, Wl2, Wr2, att2, b2,
           obs_W1, obs_b1, obs_W2, obs_b2, noise_W, noise_b,
           fus_W1, fus_b1, fus_W2, fus_b2):
    loop = jnp.arange(N, dtype=edge_index.dtype)
    pad = jnp.zeros((EP - E_TOT,), edge_index.dtype)
    src = jnp.concatenate([edge_index[0], loop, pad])
    dst = jnp.concatenate([edge_index[1], loop, pad])

    xl1, xr1 = _tc_dual_matmul(x, Wl1, Wr1, 128, 128)
    p1, d1 = _sc_edge_pass(_pack2(xl1), _pack2(xr1), src, dst,
                           _pack2(att1), 128)
    xl2, xr2 = _tc_norm_elu_matmul(p1, d1, b1, Wl2, Wr2, 128, 32)
    p2, d2 = _sc_edge_pass(_pack2(xl2), _pack2(xr2), src, dst,
                           _pack2(att2), 32)
    obs = observable_features.reshape(B, 5)
    return _tc_final(p2, d2, b2, batch, obs, noise_factor, noisy_exp,
                     obs_W1, obs_b1, obs_W2, obs_b2, noise_W, noise_b,
                     fus_W1, fus_b1, fus_W2, fus_b2)


# speedup vs baseline: 18.7302x; 1.0182x over previous
import functools

import jax
import jax.numpy as jnp
from jax import lax
from jax.experimental import pallas as pl
from jax.experimental.pallas import tpu as pltpu
from jax.experimental.pallas import tpu_sc as plsc

N = 10000
B = 64
E_TOT = 330000
NC, NS, LANES = 2, 16, 16
NW = NC * NS
EP = 331776
PER_W = EP // NW
C = 64
NCH = PER_W // C
NG = C // LANES
ROWS_T = 624

F32 = jnp.float32
I32 = jnp.int32


def _pack2(x):
    h = x.astype(jnp.bfloat16)
    return lax.bitcast_convert_type(
        h.reshape(*h.shape[:-1], h.shape[-1] // 2, 2), I32)


def _sc_edge_pass(xl, xr, src, dst, att, D):
    W = D // 2
    mesh = plsc.VectorSubcoreMesh(
        core_axis_name="c", subcore_axis_name="s",
        num_cores=NC, num_subcores=NS)

    @functools.partial(
        pl.kernel,
        out_type=[jax.ShapeDtypeStruct((NC, N, D), F32),
                  jax.ShapeDtypeStruct((NC, N), F32)],
        mesh=mesh,
        compiler_params=pltpu.CompilerParams(
            needs_layout_passes=False,
            use_tc_tiling_on_sc=False),
        scratch_types=[
            pltpu.VMEM_SHARED((N, D), F32),
            pltpu.VMEM_SHARED((N,), F32),
            pltpu.VMEM((2, C), I32),
            pltpu.VMEM((2, C), I32),
            pltpu.VMEM((2, C, W), I32),
            pltpu.VMEM((2, C, W), I32),
            pltpu.VMEM((2, C, D), F32),
            pltpu.VMEM((2, C), F32),
            pltpu.VMEM((2, C), I32),
            pltpu.VMEM((W,), I32),
            [pltpu.SemaphoreType.DMA] * 2,
            [pltpu.SemaphoreType.DMA] * 2,
            [pltpu.SemaphoreType.DMA] * 2,
        ],
    )
    def k(xl_h, xr_h, src_h, dst_h, att_h, out_h, den_h,
          acc_sh, den_sh, ibs, ibd, bufL, bufR, bufS, exb, sidx, att_v,
          semG, semS, semI):
        cid = lax.axis_index("c")
        sid = lax.axis_index("s")
        wid = sid * NC + cid
        zero16 = jnp.zeros((LANES,), F32)
        iota = lax.iota(I32, LANES)
        rows = [g * LANES + iota for g in range(NG)]
        cbase = wid * NCH

        for g in range(NG):
            exb[0, pl.ds(g * LANES, LANES)] = zero16

        def zb(r, c):
            for ko in range(D // LANES):
                bufS[0, r, pl.ds(ko * LANES, LANES)] = zero16
            return c
        lax.fori_loop(0, C, zb, 0)

        r0 = sid * ROWS_T
        for off in range(0, ROWS_T, C):
            r = min(C, ROWS_T - off)
            pltpu.sync_copy(bufS.at[0, pl.ds(0, r)],
                            acc_sh.at[pl.ds(r0 + off, r)])

        @pl.when(sid == NS - 1)
        def _ztail():
            pltpu.sync_copy(bufS.at[0, pl.ds(0, 16)],
                            acc_sh.at[pl.ds(N - 16, 16)])

        d0 = sid * 640
        for kk in range(10):
            off = kk * 64
            if kk < 6:
                pltpu.sync_copy(exb.at[0], den_sh.at[pl.ds(d0 + off, 64)])
            else:
                @pl.when(sid < NS - 1)
                def _zm(off=off):
                    pltpu.sync_copy(exb.at[0],
                                    den_sh.at[pl.ds(d0 + off, 64)])
        @pl.when(sid == NS - 1)
        def _zt():
            pltpu.sync_copy(exb.at[0, pl.ds(0, 16)],
                            den_sh.at[pl.ds(N - 16, 16)])

        pltpu.sync_copy(att_h, att_v)
        plsc.subcore_barrier()

        def issue_idx(i, p):
            pltpu.async_copy(src_h.at[cbase + i], ibs.at[p], semI[p])
            pltpu.async_copy(dst_h.at[cbase + i], ibd.at[p], semI[p])

        def wait_idx(p):
            pltpu.make_async_copy(src_h.at[cbase], ibs.at[p],
                                  semI[p]).wait()
            pltpu.make_async_copy(dst_h.at[cbase], ibd.at[p],
                                  semI[p]).wait()

        def issue_rows(p):
            pltpu.async_copy(xl_h.at[ibs.at[p]], bufL.at[p], semG[p])
            pltpu.async_copy(xr_h.at[ibd.at[p]], bufR.at[p], semG[p])

        def wait_rows(p):
            pltpu.make_async_copy(xl_h.at[ibs.at[p]], bufL.at[p],
                                  semG[p]).wait()
            pltpu.make_async_copy(xr_h.at[ibd.at[p]], bufR.at[p],
                                  semG[p]).wait()

        def issue_scat(p):
            pltpu.async_copy(bufS.at[p], acc_sh.at[sidx.at[p]],
                             semS[p], add=True)
            pltpu.async_copy(exb.at[p], den_sh.at[sidx.at[p]],
                             semS[p], add=True)

        def wait_scat(p):
            pltpu.make_async_copy(bufS.at[p], acc_sh.at[sidx.at[p]],
                                  semS[p]).wait()
            pltpu.make_async_copy(exb.at[p], den_sh.at[sidx.at[p]],
                                  semS[p]).wait()

        def do_chunk(i, p, wait_i, wait_s):
            wait_rows(p)
            if wait_s:
                wait_scat(p)
            for g in range(NG):
                sidx[p, pl.ds(g * LANES, LANES)] = (
                    ibd[p, pl.ds(g * LANES, LANES)])

            @pl.when(i + 2 < NCH)
            def _():
                issue_idx(i + 2, p)
            if wait_i:
                @pl.when(i + 1 < NCH)
                def _():
                    wait_idx(1 - p)
                    issue_rows(1 - p)
            else:
                issue_rows(1 - p)

            def _up(x):
                return plsc.unpack(
                    plsc.bitcast(x, jnp.bfloat16),
                    format=plsc.PackFormat.INTERLEAVED,
                    preferred_element_type=F32)

            def fA(w, accs):
                rot = (w + iota) & (W - 1)
                a0, a1 = _up(plsc.load_gather(att_v, [rot]))
                out = []
                for g in range(NG):
                    l0, l1 = _up(plsc.load_gather(bufL.at[p],
                                                  [rows[g], rot]))
                    r0, r1 = _up(plsc.load_gather(bufR.at[p],
                                                  [rows[g], rot]))
                    h0 = l0 + r0
                    h1 = l1 + r1
                    h0 = jnp.maximum(h0, 0.2 * h0)
                    h1 = jnp.maximum(h1, 0.2 * h1)
                    out.append(accs[g] + h0 * a0 + h1 * a1)
                return tuple(out)
            accs = lax.fori_loop(0, W, fA,
                                 tuple(zero16 for _ in range(NG)),
                                 unroll=4)

            base_e = (cbase + i) * C
            for g in range(NG):
                gid = base_e + rows[g]
                ex = jnp.where(gid < E_TOT, jnp.exp(accs[g]), 0.0)
                exb[p, pl.ds(g * LANES, LANES)] = ex

            exs = [exb[p, pl.ds(g * LANES, LANES)] for g in range(NG)]

            def fB(w, c):
                rot = (w + iota) & (W - 1)
                f0 = rot * 2
                for g in range(NG):
                    v0, v1 = _up(plsc.load_gather(bufL.at[p],
                                                  [rows[g], rot]))
                    plsc.store_scatter(bufS.at[p], [rows[g], f0],
                                       v0 * exs[g])
                    plsc.store_scatter(bufS.at[p], [rows[g], f0 + 1],
                                       v1 * exs[g])
                return c
            lax.fori_loop(0, W, fB, 0, unroll=4)
            issue_scat(p)

        for p in (0, 1):
            issue_idx(p, p)
            wait_idx(p)
        issue_rows(0)
        do_chunk(0, 0, wait_i=False, wait_s=False)
        do_chunk(1, 1, wait_i=True, wait_s=False)

        def pair(j, c):
            do_chunk(2 * j, 0, wait_i=True, wait_s=True)
            do_chunk(2 * j + 1, 1, wait_i=True, wait_s=True)
            return c
        lax.fori_loop(1, NCH // 2, pair, 0)

        wait_scat(0)
        wait_scat(1)

        plsc.subcore_barrier()

        pltpu.sync_copy(acc_sh.at[pl.ds(r0, ROWS_T)],
                        out_h.at[cid, pl.ds(r0, ROWS_T)])

        @pl.when(sid == NS - 1)
        def _wtail():
            pltpu.sync_copy(acc_sh.at[pl.ds(N - 16, 16)],
                            out_h.at[cid, pl.ds(N - 16, 16)])

        @pl.when(sid == 0)
        def _wd():
            pltpu.sync_copy(den_sh, den_h.at[cid])

    return k(xl, xr, src.reshape(EP // C, C), dst.reshape(EP // C, C),
             att)


def _tc_dual_matmul(x, Wl, Wr, K, M):
    R = 400

    def body(x_ref, wl_ref, wr_ref, ol_ref, or_ref):
        xb = x_ref[...]
        ol_ref[...] = jnp.dot(xb, wl_ref[...], preferred_element_type=F32)
        or_ref[...] = jnp.dot(xb, wr_ref[...], preferred_element_type=F32)

    return pl.pallas_call(
        body,
        grid=(N // R,),
        in_specs=[pl.BlockSpec((R, K), lambda i: (i, 0)),
                  pl.BlockSpec((K, M), lambda i: (0, 0)),
                  pl.BlockSpec((K, M), lambda i: (0, 0))],
        out_specs=[pl.BlockSpec((R, M), lambda i: (i, 0)),
                   pl.BlockSpec((R, M), lambda i: (i, 0))],
        out_shape=[jax.ShapeDtypeStruct((N, M), F32),
                   jax.ShapeDtypeStruct((N, M), F32)],
    )(x, Wl, Wr)


def _tc_norm_elu_matmul(p, den, bias, Wl, Wr, K, M):
    R = 400
    den3 = den.reshape(NC, N, 1)
    b2 = bias.reshape(1, K)

    def body(p_ref, d_ref, b_ref, wl_ref, wr_ref, ol_ref, or_ref):
        h = p_ref[0] + p_ref[1]
        dd = d_ref[0] + d_ref[1]
        h = h / dd + b_ref[...]
        h = jnp.where(h > 0, h, jnp.exp(jnp.minimum(h, 0.0)) - 1.0)
        ol_ref[...] = jnp.dot(h, wl_ref[...], preferred_element_type=F32)
        or_ref[...] = jnp.dot(h, wr_ref[...], preferred_element_type=F32)

    return pl.pallas_call(
        body,
        grid=(N // R,),
        in_specs=[pl.BlockSpec((NC, R, K), lambda i: (0, i, 0)),
                  pl.BlockSpec((NC, R, 1), lambda i: (0, i, 0)),
                  pl.BlockSpec((1, K), lambda i: (0, 0)),
                  pl.BlockSpec((K, M), lambda i: (0, 0)),
                  pl.BlockSpec((K, M), lambda i: (0, 0))],
        out_specs=[pl.BlockSpec((R, M), lambda i: (i, 0)),
                   pl.BlockSpec((R, M), lambda i: (i, 0))],
        out_shape=[jax.ShapeDtypeStruct((N, M), F32),
                   jax.ShapeDtypeStruct((N, M), F32)],
    )(p, den3, b2, Wl, Wr)


def _tc_final(p2, den2, b2, batch, obs, nf, ne,
              oW1, ob1, oW2, ob2, nW, nb, fW1, fb1, fW2, fb2):
    den3 = den2.reshape(NC, N, 1)
    bt2 = batch.reshape(N, 1)
    fa, fb, fc, fd = fW1[:32], fW1[32:40], fW1[40:44], fW1[44:45]

    def body(p_ref, d_ref, b_ref, bt_ref, obs_ref, nf_ref, ne_ref,
             ow1, ob1r, ow2, ob2r, nwr, nbr,
             far, fbr, fcr, fdr, fb1r, fw2r, fb2r, o_ref):
        h = p_ref[0] + p_ref[1]
        dd = d_ref[0] + d_ref[1]
        h = h / dd + b_ref[...]
        h = jnp.where(h > 0, h, jnp.exp(jnp.minimum(h, 0.0)) - 1.0)
        bt = bt_ref[...]
        seg = lax.broadcasted_iota(I32, (B, N), 0)
        oh = (seg == bt.reshape(1, N)).astype(F32)
        sums = jnp.dot(oh, h, preferred_element_type=F32)
        cnt = jnp.sum(oh, axis=1, keepdims=True)
        ce = sums / jnp.maximum(cnt, 1.0)
        obs_e = jnp.dot(
            jnp.maximum(jnp.dot(obs_ref[...], ow1[...],
                                preferred_element_type=F32) + ob1r[...], 0.0),
            ow2[...], preferred_element_type=F32) + ob2r[...]
        noi = jnp.dot(nf_ref[...], nwr[...],
                      preferred_element_type=F32) + nbr[...]
        z = (jnp.dot(ce, far[...], preferred_element_type=F32)
             + jnp.dot(obs_e, fbr[...], preferred_element_type=F32)
             + jnp.dot(noi, fcr[...], preferred_element_type=F32)
             + jnp.dot(ne_ref[...], fdr[...], preferred_element_type=F32)
             + fb1r[...])
        corr = jnp.dot(jnp.maximum(z, 0.0), fw2r[...],
                       preferred_element_type=F32) + fb2r[...]
        o_ref[...] = ne_ref[...] + corr

    args = (p2, den3, b2.reshape(1, 32), bt2, obs, nf, ne,
            oW1, ob1.reshape(1, 32), oW2, ob2.reshape(1, 8),
            nW, nb.reshape(1, 4),
            fa, fb, fc, fd, fb1.reshape(1, 256), fW2, fb2.reshape(1, 1))
    return pl.pallas_call(
        body,
        out_shape=jax.ShapeDtypeStruct((B, 1), F32),
    )(*args)


def kernel(x, edge_index, batch, observable_features, noise_factor,
           noisy_exp, Wl1, Wr1, att1, b1, Wl2, Wr2, att2, b2,
           obs_W1, obs_b1, obs_W2, obs_b2, noise_W, noise_b,
           fus_W1, fus_b1, fus_W2, fus_b2):
    loop = jnp.arange(N, dtype=edge_index.dtype)
    pad = jnp.zeros((EP - E_TOT,), edge_index.dtype)
    src = jnp.concatenate([edge_index[0], loop, pad])
    dst = jnp.concatenate([edge_index[1], loop, pad])

    xl1, xr1 = _tc_dual_matmul(x, Wl1, Wr1, 128, 128)
    p1, d1 = _sc_edge_pass(_pack2(xl1), _pack2(xr1), src, dst,
                           _pack2(att1), 128)
    xl2, xr2 = _tc_norm_elu_matmul(p1, d1, b1, Wl2, Wr2, 128, 32)
    p2, d2 = _sc_edge_pass(_pack2(xl2), _pack2(xr2), src, dst,
                           _pack2(att2), 32)
    obs = observable_features.reshape(B, 5)
    return _tc_final(p2, d2, b2, batch, obs, noise_factor, noisy_exp,
                     obs_W1, obs_b1, obs_W2, obs_b2, noise_W, noise_b,
                     fus_W1, fus_b1, fus_W2, fus_b2)


# speedup vs baseline: 19.6563x; 1.0494x over previous
import functools

import jax
import jax.numpy as jnp
from jax import lax
from jax.experimental import pallas as pl
from jax.experimental.pallas import tpu as pltpu
from jax.experimental.pallas import tpu_sc as plsc

N = 10000
B = 64
E_TOT = 330000
NC, NS, LANES = 2, 16, 16
NW = NC * NS
EP = 331776
PER_W = EP // NW
C = 64
NCH = PER_W // C
NG = C // LANES
ROWS_T = 624

F32 = jnp.float32
I32 = jnp.int32


def _pack2(x):
    h = x.astype(jnp.bfloat16)
    return lax.bitcast_convert_type(
        h.reshape(*h.shape[:-1], h.shape[-1] // 2, 2), I32)


def _sc_edge_pass(xl, xr, src, dst, att, D):
    W = D // 2
    mesh = plsc.VectorSubcoreMesh(
        core_axis_name="c", subcore_axis_name="s",
        num_cores=NC, num_subcores=NS)

    @functools.partial(
        pl.kernel,
        out_type=[jax.ShapeDtypeStruct((NC, N, D), F32),
                  jax.ShapeDtypeStruct((NC, N), F32)],
        mesh=mesh,
        compiler_params=pltpu.CompilerParams(
            needs_layout_passes=False,
            use_tc_tiling_on_sc=False),
        scratch_types=[
            pltpu.VMEM_SHARED((N, D), F32),
            pltpu.VMEM_SHARED((N,), F32),
            pltpu.VMEM((2, C), I32),
            pltpu.VMEM((2, C), I32),
            pltpu.VMEM((2, C, W), I32),
            pltpu.VMEM((2, C, W), I32),
            pltpu.VMEM((2, C, D), F32),
            pltpu.VMEM((2, C), F32),
            pltpu.VMEM((2, C), I32),
            pltpu.VMEM((W,), I32),
            [pltpu.SemaphoreType.DMA] * 2,
            [pltpu.SemaphoreType.DMA] * 2,
            [pltpu.SemaphoreType.DMA] * 2,
        ],
    )
    def k(xl_h, xr_h, src_h, dst_h, att_h, out_h, den_h,
          acc_sh, den_sh, ibs, ibd, bufL, bufR, bufS, exb, sidx, att_v,
          semG, semS, semI):
        cid = lax.axis_index("c")
        sid = lax.axis_index("s")
        wid = sid * NC + cid
        zero16 = jnp.zeros((LANES,), F32)
        iota = lax.iota(I32, LANES)
        rows = [g * LANES + iota for g in range(NG)]
        cbase = wid * NCH

        for g in range(NG):
            exb[0, pl.ds(g * LANES, LANES)] = zero16

        def zb(r, c):
            for ko in range(D // LANES):
                bufS[0, r, pl.ds(ko * LANES, LANES)] = zero16
            return c
        lax.fori_loop(0, C, zb, 0)

        r0 = sid * ROWS_T
        for off in range(0, ROWS_T, C):
            r = min(C, ROWS_T - off)
            pltpu.sync_copy(bufS.at[0, pl.ds(0, r)],
                            acc_sh.at[pl.ds(r0 + off, r)])

        @pl.when(sid == NS - 1)
        def _ztail():
            pltpu.sync_copy(bufS.at[0, pl.ds(0, 16)],
                            acc_sh.at[pl.ds(N - 16, 16)])

        d0 = sid * 640
        for kk in range(10):
            off = kk * 64
            if kk < 6:
                pltpu.sync_copy(exb.at[0], den_sh.at[pl.ds(d0 + off, 64)])
            else:
                @pl.when(sid < NS - 1)
                def _zm(off=off):
                    pltpu.sync_copy(exb.at[0],
                                    den_sh.at[pl.ds(d0 + off, 64)])
        @pl.when(sid == NS - 1)
        def _zt():
            pltpu.sync_copy(exb.at[0, pl.ds(0, 16)],
                            den_sh.at[pl.ds(N - 16, 16)])

        pltpu.sync_copy(att_h, att_v)
        plsc.subcore_barrier()

        def issue_idx(i, p):
            pltpu.async_copy(src_h.at[cbase + i], ibs.at[p], semI[p])
            pltpu.async_copy(dst_h.at[cbase + i], ibd.at[p], semI[p])

        def wait_idx(p):
            pltpu.make_async_copy(src_h.at[cbase], ibs.at[p],
                                  semI[p]).wait()
            pltpu.make_async_copy(dst_h.at[cbase], ibd.at[p],
                                  semI[p]).wait()

        def issue_rows(p):
            pltpu.async_copy(xl_h.at[ibs.at[p]], bufL.at[p], semG[p])
            pltpu.async_copy(xr_h.at[ibd.at[p]], bufR.at[p], semG[p])

        def wait_rows(p):
            pltpu.make_async_copy(xl_h.at[ibs.at[p]], bufL.at[p],
                                  semG[p]).wait()
            pltpu.make_async_copy(xr_h.at[ibd.at[p]], bufR.at[p],
                                  semG[p]).wait()

        def issue_scat(p):
            pltpu.async_copy(bufS.at[p], acc_sh.at[sidx.at[p]],
                             semS[p], add=True)
            pltpu.async_copy(exb.at[p], den_sh.at[sidx.at[p]],
                             semS[p], add=True)

        def wait_scat(p):
            pltpu.make_async_copy(bufS.at[p], acc_sh.at[sidx.at[p]],
                                  semS[p]).wait()
            pltpu.make_async_copy(exb.at[p], den_sh.at[sidx.at[p]],
                                  semS[p]).wait()

        def do_chunk(i, p, wait_i, wait_s):
            wait_rows(p)
            if wait_s:
                wait_scat(p)
            for g in range(NG):
                sidx[p, pl.ds(g * LANES, LANES)] = (
                    ibd[p, pl.ds(g * LANES, LANES)])

            @pl.when(i + 2 < NCH)
            def _():
                issue_idx(i + 2, p)
            if wait_i:
                @pl.when(i + 1 < NCH)
                def _():
                    wait_idx(1 - p)
                    issue_rows(1 - p)
            else:
                issue_rows(1 - p)

            def _up2(xb):
                return plsc.unpack(
                    xb, format=plsc.PackFormat.INTERLEAVED,
                    preferred_element_type=F32)

            def _up(x):
                return _up2(plsc.bitcast(x, jnp.bfloat16))

            def fA(w, accs):
                rot = (w + iota) & (W - 1)
                ab = plsc.bitcast(plsc.load_gather(att_v, [rot]),
                                  jnp.bfloat16)
                out = []
                for g in range(NG):
                    lb = plsc.bitcast(
                        plsc.load_gather(bufL.at[p], [rows[g], rot]),
                        jnp.bfloat16)
                    rb = plsc.bitcast(
                        plsc.load_gather(bufR.at[p], [rows[g], rot]),
                        jnp.bfloat16)
                    h = lb + rb
                    h = jnp.maximum(h, 0.2 * h)
                    p0, p1 = _up2(h * ab)
                    out.append(accs[g] + (p0 + p1))
                return tuple(out)
            accs = lax.fori_loop(0, W, fA,
                                 tuple(zero16 for _ in range(NG)),
                                 unroll=4)

            base_e = (cbase + i) * C
            for g in range(NG):
                gid = base_e + rows[g]
                ex = jnp.where(gid < E_TOT, jnp.exp(accs[g]), 0.0)
                exb[p, pl.ds(g * LANES, LANES)] = ex

            exs = [exb[p, pl.ds(g * LANES, LANES)] for g in range(NG)]

            expk = [plsc.pack(exs[g], exs[g],
                              format=plsc.PackFormat.INTERLEAVED)
                    for g in range(NG)]

            def fB(w, c):
                rot = (w + iota) & (W - 1)
                f0 = rot * 2
                for g in range(NG):
                    vb = plsc.bitcast(
                        plsc.load_gather(bufL.at[p], [rows[g], rot]),
                        jnp.bfloat16)
                    v0, v1 = _up2(vb * expk[g])
                    plsc.store_scatter(bufS.at[p], [rows[g], f0], v0)
                    plsc.store_scatter(bufS.at[p], [rows[g], f0 + 1], v1)
                return c
            lax.fori_loop(0, W, fB, 0, unroll=4)
            issue_scat(p)

        for p in (0, 1):
            issue_idx(p, p)
            wait_idx(p)
        issue_rows(0)
        do_chunk(0, 0, wait_i=False, wait_s=False)
        do_chunk(1, 1, wait_i=True, wait_s=False)

        def pair(j, c):
            do_chunk(2 * j, 0, wait_i=True, wait_s=True)
            do_chunk(2 * j + 1, 1, wait_i=True, wait_s=True)
            return c
        lax.fori_loop(1, NCH // 2, pair, 0)

        wait_scat(0)
        wait_scat(1)

        plsc.subcore_barrier()

        pltpu.sync_copy(acc_sh.at[pl.ds(r0, ROWS_T)],
                        out_h.at[cid, pl.ds(r0, ROWS_T)])

        @pl.when(sid == NS - 1)
        def _wtail():
            pltpu.sync_copy(acc_sh.at[pl.ds(N - 16, 16)],
                            out_h.at[cid, pl.ds(N - 16, 16)])

        @pl.when(sid == 0)
        def _wd():
            pltpu.sync_copy(den_sh, den_h.at[cid])

    return k(xl, xr, src.reshape(EP // C, C), dst.reshape(EP // C, C),
             att)


def _tc_dual_matmul(x, Wl, Wr, K, M):
    R = 400

    def body(x_ref, wl_ref, wr_ref, ol_ref, or_ref):
        xb = x_ref[...]
        ol_ref[...] = jnp.dot(xb, wl_ref[...], preferred_element_type=F32)
        or_ref[...] = jnp.dot(xb, wr_ref[...], preferred_element_type=F32)

    return pl.pallas_call(
        body,
        grid=(N // R,),
        in_specs=[pl.BlockSpec((R, K), lambda i: (i, 0)),
                  pl.BlockSpec((K, M), lambda i: (0, 0)),
                  pl.BlockSpec((K, M), lambda i: (0, 0))],
        out_specs=[pl.BlockSpec((R, M), lambda i: (i, 0)),
                   pl.BlockSpec((R, M), lambda i: (i, 0))],
        out_shape=[jax.ShapeDtypeStruct((N, M), F32),
                   jax.ShapeDtypeStruct((N, M), F32)],
    )(x, Wl, Wr)


def _tc_norm_elu_matmul(p, den, bias, Wl, Wr, K, M):
    R = 400
    den3 = den.reshape(NC, N, 1)
    b2 = bias.reshape(1, K)

    def body(p_ref, d_ref, b_ref, wl_ref, wr_ref, ol_ref, or_ref):
        h = p_ref[0] + p_ref[1]
        dd = d_ref[0] + d_ref[1]
        h = h / dd + b_ref[...]
        h = jnp.where(h > 0, h, jnp.exp(jnp.minimum(h, 0.0)) - 1.0)
        ol_ref[...] = jnp.dot(h, wl_ref[...], preferred_element_type=F32)
        or_ref[...] = jnp.dot(h, wr_ref[...], preferred_element_type=F32)

    return pl.pallas_call(
        body,
        grid=(N // R,),
        in_specs=[pl.BlockSpec((NC, R, K), lambda i: (0, i, 0)),
                  pl.BlockSpec((NC, R, 1), lambda i: (0, i, 0)),
                  pl.BlockSpec((1, K), lambda i: (0, 0)),
                  pl.BlockSpec((K, M), lambda i: (0, 0)),
                  pl.BlockSpec((K, M), lambda i: (0, 0))],
        out_specs=[pl.BlockSpec((R, M), lambda i: (i, 0)),
                   pl.BlockSpec((R, M), lambda i: (i, 0))],
        out_shape=[jax.ShapeDtypeStruct((N, M), F32),
                   jax.ShapeDtypeStruct((N, M), F32)],
    )(p, den3, b2, Wl, Wr)


def _tc_final(p2, den2, b2, batch, obs, nf, ne,
              oW1, ob1, oW2, ob2, nW, nb, fW1, fb1, fW2, fb2):
    den3 = den2.reshape(NC, N, 1)
    bt2 = batch.reshape(N, 1)
    fa, fb, fc, fd = fW1[:32], fW1[32:40], fW1[40:44], fW1[44:45]

    def body(p_ref, d_ref, b_ref, bt_ref, obs_ref, nf_ref, ne_ref,
             ow1, ob1r, ow2, ob2r, nwr, nbr,
             far, fbr, fcr, fdr, fb1r, fw2r, fb2r, o_ref):
        h = p_ref[0] + p_ref[1]
        dd = d_ref[0] + d_ref[1]
        h = h / dd + b_ref[...]
        h = jnp.where(h > 0, h, jnp.exp(jnp.minimum(h, 0.0)) - 1.0)
        bt = bt_ref[...]
        seg = lax.broadcasted_iota(I32, (B, N), 0)
        oh = (seg == bt.reshape(1, N)).astype(F32)
        sums = jnp.dot(oh, h, preferred_element_type=F32)
        cnt = jnp.sum(oh, axis=1, keepdims=True)
        ce = sums / jnp.maximum(cnt, 1.0)
        obs_e = jnp.dot(
            jnp.maximum(jnp.dot(obs_ref[...], ow1[...],
                                preferred_element_type=F32) + ob1r[...], 0.0),
            ow2[...], preferred_element_type=F32) + ob2r[...]
        noi = jnp.dot(nf_ref[...], nwr[...],
                      preferred_element_type=F32) + nbr[...]
        z = (jnp.dot(ce, far[...], preferred_element_type=F32)
             + jnp.dot(obs_e, fbr[...], preferred_element_type=F32)
             + jnp.dot(noi, fcr[...], preferred_element_type=F32)
             + jnp.dot(ne_ref[...], fdr[...], preferred_element_type=F32)
             + fb1r[...])
        corr = jnp.dot(jnp.maximum(z, 0.0), fw2r[...],
                       preferred_element_type=F32) + fb2r[...]
        o_ref[...] = ne_ref[...] + corr

    args = (p2, den3, b2.reshape(1, 32), bt2, obs, nf, ne,
            oW1, ob1.reshape(1, 32), oW2, ob2.reshape(1, 8),
            nW, nb.reshape(1, 4),
            fa, fb, fc, fd, fb1.reshape(1, 256), fW2, fb2.reshape(1, 1))
    return pl.pallas_call(
        body,
        out_shape=jax.ShapeDtypeStruct((B, 1), F32),
    )(*args)


def kernel(x, edge_index, batch, observable_features, noise_factor,
           noisy_exp, Wl1, Wr1, att1, b1, Wl2, Wr2, att2, b2,
           obs_W1, obs_b1, obs_W2, obs_b2, noise_W, noise_b,
           fus_W1, fus_b1, fus_W2, fus_b2):
    loop = jnp.arange(N, dtype=edge_index.dtype)
    pad = jnp.zeros((EP - E_TOT,), edge_index.dtype)
    src = jnp.concatenate([edge_index[0], loop, pad])
    dst = jnp.concatenate([edge_index[1], loop, pad])

    xl1, xr1 = _tc_dual_matmul(x, Wl1, Wr1, 128, 128)
    p1, d1 = _sc_edge_pass(_pack2(xl1), _pack2(xr1), src, dst,
                           _pack2(att1), 128)
    xl2, xr2 = _tc_norm_elu_matmul(p1, d1, b1, Wl2, Wr2, 128, 32)
    p2, d2 = _sc_edge_pass(_pack2(xl2), _pack2(xr2), src, dst,
                           _pack2(att2), 32)
    obs = observable_features.reshape(B, 5)
    return _tc_final(p2, d2, b2, batch, obs, noise_factor, noisy_exp,
                     obs_W1, obs_b1, obs_W2, obs_b2, noise_W, noise_b,
                     fus_W1, fus_b1, fus_W2, fus_b2)
